# Initial kernel scaffold; baseline (speedup 1.0000x reference)
import jax
import jax.numpy as jnp
from jax.experimental import pallas as pl


def kernel(x, edge_index, edge_attr, params):
    raise NotImplementedError("write your pallas kernel here")


# speedup vs baseline: 2.3676x; 2.3676x over previous
import functools

import jax
import jax.numpy as jnp
from jax import lax
from jax.experimental import pallas as pl
from jax.experimental.pallas import tpu as pltpu
from jax.experimental.pallas import tpu_sc as plsc

N_NODES = 10000
N_EDGES = 320000
DN = 128
DE = 16

NC, NS = 2, 16
NW = NC * NS
E_PER_TILE = N_EDGES // NW
ROW = 100
CH_ROWS = 4
CH = ROW * CH_ROWS
N_CH = E_PER_TILE // CH
MROW = 80
M_CH = E_PER_TILE // MROW
N_PAD = 10240
NODES_PER_TILE = N_PAD // NS
ZR = 64

_mesh = plsc.VectorSubcoreMesh(core_axis_name="c", subcore_axis_name="s")


@functools.partial(
    pl.kernel,
    out_type=jax.ShapeDtypeStruct((NC, N_PAD, DN), jnp.float32),
    mesh=_mesh,
    scratch_types=[
        pltpu.VMEM((1, MROW), jnp.int32),
        pltpu.VMEM((1, MROW), jnp.int32),
        pltpu.VMEM((MROW, DN), jnp.float32),
        pltpu.VMEM((MROW, DN), jnp.float32),
        pltpu.VMEM_SHARED((N_PAD, DN), jnp.float32),
        pltpu.SemaphoreType.DMA,
    ],
)
def _msgpass(src_hbm, dst_hbm, ea_hbm, x_hbm, out_hbm,
             idx_s, idx_d, xg, eb, aggr, sem):
    c = lax.axis_index("c")
    s = lax.axis_index("s")
    wid = c * NS + s

    zvec = jnp.zeros((16,), jnp.float32)

    def _zrow(i, _):
        for j in range(DN // 16):
            eb[i, pl.ds(j * 16, 16)] = zvec
        return 0

    lax.fori_loop(0, ZR, _zrow, 0)
    my_node0 = s * NODES_PER_TILE
    for k in range(NODES_PER_TILE // ZR):
        pltpu.sync_copy(eb.at[pl.ds(0, ZR)],
                        aggr.at[pl.ds(my_node0 + k * ZR, ZR)])
    plsc.subcore_barrier()

    e0 = wid * E_PER_TILE

    def _chunk(k, _):
        pltpu.sync_copy(src_hbm.at[wid, k], idx_s)
        pltpu.sync_copy(dst_hbm.at[wid, k], idx_d)
        pltpu.sync_copy(ea_hbm.at[pl.ds(e0 + k * MROW, MROW)], eb)
        pltpu.async_copy(x_hbm.at[idx_s.at[0]], xg, sem).wait()

        def _edge(e, _):
            for j in range(DN // 16):
                sl = pl.ds(j * 16, 16)
                eb[e, sl] = jnp.maximum(eb[e, sl] + xg[e, sl], 0.0)
            return 0

        lax.fori_loop(0, MROW, _edge, 0)
        pltpu.sync_copy(eb, aggr.at[idx_d.at[0]], add=True)
        return 0

    lax.fori_loop(0, M_CH, _chunk, 0)
    plsc.subcore_barrier()

    for k in range(NODES_PER_TILE // ZR):
        r = my_node0 + k * ZR
        pltpu.sync_copy(aggr.at[pl.ds(r, ZR)], eb.at[pl.ds(0, ZR)])
        pltpu.sync_copy(eb.at[pl.ds(0, ZR)], out_hbm.at[c, pl.ds(r, ZR)])


@functools.partial(
    pl.kernel,
    out_type=jax.ShapeDtypeStruct((N_EDGES, DE), jnp.float32),
    mesh=_mesh,
    scratch_types=[
        pltpu.VMEM((CH_ROWS, ROW), jnp.int32),
        pltpu.VMEM((CH_ROWS, ROW), jnp.int32),
        pltpu.VMEM((CH, DE), jnp.float32),
        pltpu.VMEM((CH, DE), jnp.float32),
        pltpu.VMEM((CH, DE), jnp.float32),
        pltpu.SemaphoreType.DMA,
    ],
    compiler_params=pltpu.CompilerParams(use_tc_tiling_on_sc=False),
)
def _edgegather(src_hbm, dst_hbm, p_hbm, q_hbm, r_hbm, t_hbm,
                idx_s, idx_d, pg, qg, rb, sem):
    c = lax.axis_index("c")
    s = lax.axis_index("s")
    wid = c * NS + s
    e0 = wid * E_PER_TILE

    def _chunk(k, _):
        pltpu.sync_copy(src_hbm.at[wid, k], idx_s)
        pltpu.sync_copy(dst_hbm.at[wid, k], idx_d)
        pltpu.sync_copy(r_hbm.at[pl.ds(e0 + k * CH, CH)], rb)
        cps = []
        for j in range(CH_ROWS):
            cps.append(pltpu.async_copy(p_hbm.at[idx_s.at[j]],
                                        pg.at[pl.ds(j * ROW, ROW)], sem))
            cps.append(pltpu.async_copy(q_hbm.at[idx_d.at[j]],
                                        qg.at[pl.ds(j * ROW, ROW)], sem))
        for cp in cps:
            cp.wait()

        def _edge(e, _):
            rb[e] = jnp.maximum(rb[e] + pg[e] + qg[e], 0.0)
            return 0

        lax.fori_loop(0, CH, _edge, 0)
        pltpu.sync_copy(rb, t_hbm.at[pl.ds(e0 + k * CH, CH)])
        return 0

    lax.fori_loop(0, N_CH, _chunk, 0)


_EBLK = 2000
_EGRID = N_EDGES // _EBLK


def _edense_body(e_ref, w0_ref, b0_ref, w1_ref, b1_ref, wc_ref, bc_ref,
                 ea0_ref, ea1_ref, r_ref):
    e = e_ref[...]
    ea0_ref[...] = jnp.dot(e, w0_ref[...],
                           preferred_element_type=jnp.float32) + b0_ref[...]
    ea1_ref[...] = jnp.dot(e, w1_ref[...],
                           preferred_element_type=jnp.float32) + b1_ref[...]
    r_ref[...] = jnp.dot(e, wc_ref[...],
                         preferred_element_type=jnp.float32) + bc_ref[...]


def _edense(edge_attr, w0, b0, w1, b1, wc, bc):
    full = lambda shp: pl.BlockSpec(shp, lambda i: (0, 0))
    return pl.pallas_call(
        _edense_body,
        grid=(_EGRID,),
        in_specs=[
            pl.BlockSpec((_EBLK, DE), lambda i: (i, 0)),
            full((DE, DN)), full((1, DN)),
            full((DE, DN)), full((1, DN)),
            full((DE, DE)), full((1, DE)),
        ],
        out_specs=[
            pl.BlockSpec((_EBLK, DN), lambda i: (i, 0)),
            pl.BlockSpec((_EBLK, DN), lambda i: (i, 0)),
            pl.BlockSpec((_EBLK, DE), lambda i: (i, 0)),
        ],
        out_shape=[
            jax.ShapeDtypeStruct((N_EDGES, DN), jnp.float32),
            jax.ShapeDtypeStruct((N_EDGES, DN), jnp.float32),
            jax.ShapeDtypeStruct((N_EDGES, DE), jnp.float32),
        ],
    )(edge_attr, w0, b0, w1, b1, wc, bc)


def _node_body(x_ref, a_ref, w1_ref, b1_ref, w2_ref, b2_ref, gb_ref,
               epsb_ref, wp_ref, wq_ref, xn_ref, pp_ref, qq_ref):
    x = x_ref[...]
    aggr = (a_ref[0] + a_ref[1])[:N_NODES]
    h = epsb_ref[...] * x + aggr
    h = jnp.maximum(jnp.dot(h, w1_ref[...],
                            preferred_element_type=jnp.float32) + b1_ref[...],
                    0.0)
    h = jnp.dot(h, w2_ref[...],
                preferred_element_type=jnp.float32) + b2_ref[...]
    mean = jnp.mean(h, axis=0, keepdims=True)
    cent = h - mean
    var = jnp.mean(cent * cent, axis=0, keepdims=True)
    bn = gb_ref[0:1, :] * cent * lax.rsqrt(var + 1e-5) + gb_ref[1:2, :]
    xn = (x + jnp.maximum(bn, 0.0)) * 0.5
    xn_ref[...] = xn
    pp_ref[...] = jnp.dot(xn, wp_ref[...], preferred_element_type=jnp.float32)
    qq_ref[...] = jnp.dot(xn, wq_ref[...], preferred_element_type=jnp.float32)


def _node(x, a2, w1, b1, w2, b2, gb, epsb, wp, wq):
    return pl.pallas_call(
        _node_body,
        out_shape=[
            jax.ShapeDtypeStruct((N_NODES, DN), jnp.float32),
            jax.ShapeDtypeStruct((N_NODES, DE), jnp.float32),
            jax.ShapeDtypeStruct((N_NODES, DE), jnp.float32),
        ],
    )(x, a2, w1, b1, w2, b2, gb, epsb, wp, wq)


def _eout_body(t_ref, e_ref, w_ref, b_ref, o_ref):
    mlp = jnp.dot(t_ref[...], w_ref[...],
                  preferred_element_type=jnp.float32) + b_ref[...]
    o_ref[...] = e_ref[...] + mlp * 0.5


def _eout(t, edge_attr, w, b):
    full = lambda shp: pl.BlockSpec(shp, lambda i: (0, 0))
    return pl.pallas_call(
        _eout_body,
        grid=(_EGRID,),
        in_specs=[
            pl.BlockSpec((_EBLK, DE), lambda i: (i, 0)),
            pl.BlockSpec((_EBLK, DE), lambda i: (i, 0)),
            full((DE, DE)), full((1, DE)),
        ],
        out_specs=pl.BlockSpec((_EBLK, DE), lambda i: (i, 0)),
        out_shape=jax.ShapeDtypeStruct((N_EDGES, DE), jnp.float32),
    )(t, edge_attr, w, b)


def kernel(x, edge_index, edge_attr, params):
    src_i = edge_index[0].astype(jnp.int32)
    dst_i = edge_index[1].astype(jnp.int32)
    src_m = src_i.reshape(NW, M_CH, 1, MROW)
    dst_m = dst_i.reshape(NW, M_CH, 1, MROW)
    src_g = src_i.reshape(NW, N_CH, CH_ROWS, ROW)
    dst_g = dst_i.reshape(NW, N_CH, CH_ROWS, ROW)
    p0, p1 = params[0], params[1]

    wp = p1['Wm1'][0:DN]
    wq = p1['Wm1'][DN:2 * DN]
    wc = p1['Wm1'][2 * DN:]

    ea0, ea1, rmat = _edense(
        edge_attr,
        p0['We'], p0['be'].reshape(1, DN),
        p1['We'], p1['be'].reshape(1, DN),
        wc, p1['bm1'].reshape(1, DE),
    )

    ones = jnp.ones((1, DN), jnp.float32)
    for p, ea in ((p0, ea0), (p1, ea1)):
        a2 = _msgpass(src_m, dst_m, ea, x)
        gb = jnp.stack([p['bn_gamma'], p['bn_beta']])
        epsb = (1.0 + p['eps']) * ones
        x, pp, qq = _node(x, a2, p['W1'], p['b1'].reshape(1, DN),
                          p['W2'], p['b2'].reshape(1, DN), gb, epsb, wp, wq)

    t = _edgegather(src_g, dst_g, pp, qq, rmat)
    e_out = _eout(t, edge_attr, p1['Wm2'], p1['bm2'].reshape(1, DE))
    return (x, e_out)


# speedup vs baseline: 2.5054x; 1.0582x over previous
import functools

import jax
import jax.numpy as jnp
from jax import lax
from jax.experimental import pallas as pl
from jax.experimental.pallas import tpu as pltpu
from jax.experimental.pallas import tpu_sc as plsc

N_NODES = 10000
N_EDGES = 320000
DN = 128
DE = 16

NC, NS = 2, 16
NW = NC * NS
E_PER_TILE = N_EDGES // NW
ROW = 80
CH_ROWS = 5
CH = ROW * CH_ROWS
N_CH = E_PER_TILE // CH
MROW = 80
M_CH = E_PER_TILE // MROW
N_PAD = 10240
NODES_PER_TILE = N_PAD // NS
ZR = 64

_mesh = plsc.VectorSubcoreMesh(core_axis_name="c", subcore_axis_name="s")


@functools.partial(
    pl.kernel,
    out_type=jax.ShapeDtypeStruct((NC, N_PAD, DN), jnp.float32),
    mesh=_mesh,
    scratch_types=[
        pltpu.VMEM((MROW,), jnp.int32),
        pltpu.VMEM((MROW,), jnp.int32),
        pltpu.VMEM((MROW, DN), jnp.float32),
        pltpu.VMEM((MROW, DN), jnp.float32),
        pltpu.VMEM_SHARED((N_PAD, DN), jnp.float32),
        pltpu.SemaphoreType.DMA,
    ],
)
def _msgpass(src_hbm, dst_hbm, ea_hbm, x_hbm, out_hbm,
             idx_s, idx_d, xg, eb, aggr, sem):
    c = lax.axis_index("c")
    s = lax.axis_index("s")
    wid = c * NS + s

    zvec = jnp.zeros((16,), jnp.float32)

    def _zrow(i, _):
        for j in range(DN // 16):
            eb[i, pl.ds(j * 16, 16)] = zvec
        return 0

    lax.fori_loop(0, ZR, _zrow, 0)
    my_node0 = s * NODES_PER_TILE
    for k in range(NODES_PER_TILE // ZR):
        pltpu.sync_copy(eb.at[pl.ds(0, ZR)],
                        aggr.at[pl.ds(my_node0 + k * ZR, ZR)])
    plsc.subcore_barrier()

    e0 = wid * E_PER_TILE

    def _chunk(k, _):
        pltpu.sync_copy(src_hbm.at[pl.ds(e0 + k * MROW, MROW)], idx_s)
        pltpu.sync_copy(dst_hbm.at[pl.ds(e0 + k * MROW, MROW)], idx_d)
        pltpu.sync_copy(ea_hbm.at[pl.ds(e0 + k * MROW, MROW)], eb)
        pltpu.async_copy(x_hbm.at[idx_s], xg, sem).wait()

        def _edge(e, _):
            for j in range(DN // 16):
                sl = pl.ds(j * 16, 16)
                eb[e, sl] = jnp.maximum(eb[e, sl] + xg[e, sl], 0.0)
            return 0

        lax.fori_loop(0, MROW, _edge, 0)
        pltpu.sync_copy(eb, aggr.at[idx_d], add=True)
        return 0

    lax.fori_loop(0, M_CH, _chunk, 0)
    plsc.subcore_barrier()

    for k in range(NODES_PER_TILE // ZR):
        r = my_node0 + k * ZR
        pltpu.sync_copy(aggr.at[pl.ds(r, ZR)], eb.at[pl.ds(0, ZR)])
        pltpu.sync_copy(eb.at[pl.ds(0, ZR)], out_hbm.at[c, pl.ds(r, ZR)])


@functools.partial(
    pl.kernel,
    out_type=jax.ShapeDtypeStruct((N_EDGES, DE), jnp.float32),
    mesh=_mesh,
    scratch_types=[
        pltpu.VMEM((CH,), jnp.int32),
        pltpu.VMEM((CH,), jnp.int32),
        pltpu.VMEM((CH, DE), jnp.float32),
        pltpu.VMEM((CH, DE), jnp.float32),
        pltpu.VMEM((CH, DE), jnp.float32),
        pltpu.SemaphoreType.DMA,
    ],
    compiler_params=pltpu.CompilerParams(use_tc_tiling_on_sc=False),
)
def _edgegather(src_hbm, dst_hbm, p_hbm, q_hbm, r_hbm, t_hbm,
                idx_s, idx_d, pg, qg, rb, sem):
    c = lax.axis_index("c")
    s = lax.axis_index("s")
    wid = c * NS + s
    e0 = wid * E_PER_TILE

    def _chunk(k, _):
        pltpu.sync_copy(src_hbm.at[pl.ds(e0 + k * CH, CH)], idx_s)
        pltpu.sync_copy(dst_hbm.at[pl.ds(e0 + k * CH, CH)], idx_d)
        pltpu.sync_copy(r_hbm.at[pl.ds(e0 + k * CH, CH)], rb)
        cps = []
        for j in range(CH_ROWS):
            sl = pl.ds(j * ROW, ROW)
            cps.append(pltpu.async_copy(p_hbm.at[idx_s.at[sl]],
                                        pg.at[sl], sem))
            cps.append(pltpu.async_copy(q_hbm.at[idx_d.at[sl]],
                                        qg.at[sl], sem))
        for cp in cps:
            cp.wait()

        def _edge(e, _):
            rb[e] = jnp.maximum(rb[e] + pg[e] + qg[e], 0.0)
            return 0

        lax.fori_loop(0, CH, _edge, 0)
        pltpu.sync_copy(rb, t_hbm.at[pl.ds(e0 + k * CH, CH)])
        return 0

    lax.fori_loop(0, N_CH, _chunk, 0)


_EBLK = 8000
_EGRID = N_EDGES // _EBLK
_OBLK = 16000
_OGRID = N_EDGES // _OBLK


def _edense_body(e_ref, w0_ref, b0_ref, w1_ref, b1_ref, wc_ref, bc_ref,
                 ea0_ref, ea1_ref, r_ref):
    e = e_ref[...]
    ea0_ref[...] = jnp.dot(e, w0_ref[...],
                           preferred_element_type=jnp.float32) + b0_ref[...]
    ea1_ref[...] = jnp.dot(e, w1_ref[...],
                           preferred_element_type=jnp.float32) + b1_ref[...]
    r_ref[...] = jnp.dot(e, wc_ref[...],
                         preferred_element_type=jnp.float32) + bc_ref[...]


def _edense(edge_attr, w0, b0, w1, b1, wc, bc):
    full = lambda shp: pl.BlockSpec(shp, lambda i: (0, 0))
    return pl.pallas_call(
        _edense_body,
        grid=(_EGRID,),
        in_specs=[
            pl.BlockSpec((_EBLK, DE), lambda i: (i, 0)),
            full((DE, DN)), full((1, DN)),
            full((DE, DN)), full((1, DN)),
            full((DE, DE)), full((1, DE)),
        ],
        out_specs=[
            pl.BlockSpec((_EBLK, DN), lambda i: (i, 0)),
            pl.BlockSpec((_EBLK, DN), lambda i: (i, 0)),
            pl.BlockSpec((_EBLK, DE), lambda i: (i, 0)),
        ],
        out_shape=[
            jax.ShapeDtypeStruct((N_EDGES, DN), jnp.float32),
            jax.ShapeDtypeStruct((N_EDGES, DN), jnp.float32),
            jax.ShapeDtypeStruct((N_EDGES, DE), jnp.float32),
        ],
    )(edge_attr, w0, b0, w1, b1, wc, bc)


def _node_body(x_ref, a_ref, w1_ref, b1_ref, w2_ref, b2_ref, gb_ref,
               epsb_ref, wp_ref, wq_ref, xn_ref, pp_ref, qq_ref):
    x = x_ref[...]
    aggr = (a_ref[0] + a_ref[1])[:N_NODES]
    h = epsb_ref[...] * x + aggr
    h = jnp.maximum(jnp.dot(h, w1_ref[...],
                            preferred_element_type=jnp.float32) + b1_ref[...],
                    0.0)
    h = jnp.dot(h, w2_ref[...],
                preferred_element_type=jnp.float32) + b2_ref[...]
    mean = jnp.mean(h, axis=0, keepdims=True)
    cent = h - mean
    var = jnp.mean(cent * cent, axis=0, keepdims=True)
    bn = gb_ref[0:1, :] * cent * lax.rsqrt(var + 1e-5) + gb_ref[1:2, :]
    xn = (x + jnp.maximum(bn, 0.0)) * 0.5
    xn_ref[...] = xn
    pp_ref[...] = jnp.dot(xn, wp_ref[...], preferred_element_type=jnp.float32)
    qq_ref[...] = jnp.dot(xn, wq_ref[...], preferred_element_type=jnp.float32)


def _node(x, a2, w1, b1, w2, b2, gb, epsb, wp, wq):
    return pl.pallas_call(
        _node_body,
        out_shape=[
            jax.ShapeDtypeStruct((N_NODES, DN), jnp.float32),
            jax.ShapeDtypeStruct((N_NODES, DE), jnp.float32),
            jax.ShapeDtypeStruct((N_NODES, DE), jnp.float32),
        ],
    )(x, a2, w1, b1, w2, b2, gb, epsb, wp, wq)


def _eout_body(t_ref, e_ref, w_ref, b_ref, o_ref):
    mlp = jnp.dot(t_ref[...], w_ref[...],
                  preferred_element_type=jnp.float32) + b_ref[...]
    o_ref[...] = e_ref[...] + mlp * 0.5


def _eout(t, edge_attr, w, b):
    full = lambda shp: pl.BlockSpec(shp, lambda i: (0, 0))
    return pl.pallas_call(
        _eout_body,
        grid=(_OGRID,),
        in_specs=[
            pl.BlockSpec((_OBLK, DE), lambda i: (i, 0)),
            pl.BlockSpec((_OBLK, DE), lambda i: (i, 0)),
            full((DE, DE)), full((1, DE)),
        ],
        out_specs=pl.BlockSpec((_OBLK, DE), lambda i: (i, 0)),
        out_shape=jax.ShapeDtypeStruct((N_EDGES, DE), jnp.float32),
    )(t, edge_attr, w, b)


def kernel(x, edge_index, edge_attr, params):
    src_i = edge_index[0].astype(jnp.int32)
    dst_i = edge_index[1].astype(jnp.int32)
    p0, p1 = params[0], params[1]

    wp = p1['Wm1'][0:DN]
    wq = p1['Wm1'][DN:2 * DN]
    wc = p1['Wm1'][2 * DN:]

    ea0, ea1, rmat = _edense(
        edge_attr,
        p0['We'], p0['be'].reshape(1, DN),
        p1['We'], p1['be'].reshape(1, DN),
        wc, p1['bm1'].reshape(1, DE),
    )

    ones = jnp.ones((1, DN), jnp.float32)
    for p, ea in ((p0, ea0), (p1, ea1)):
        a2 = _msgpass(src_i, dst_i, ea, x)
        gb = jnp.stack([p['bn_gamma'], p['bn_beta']])
        epsb = (1.0 + p['eps']) * ones
        x, pp, qq = _node(x, a2, p['W1'], p['b1'].reshape(1, DN),
                          p['W2'], p['b2'].reshape(1, DN), gb, epsb, wp, wq)

    t = _edgegather(src_i, dst_i, pp, qq, rmat)
    e_out = _eout(t, edge_attr, p1['Wm2'], p1['bm2'].reshape(1, DE))
    return (x, e_out)


# speedup vs baseline: 3.4463x; 1.3756x over previous
import functools

import jax
import jax.numpy as jnp
from jax import lax
from jax.experimental import pallas as pl
from jax.experimental.pallas import tpu as pltpu
from jax.experimental.pallas import tpu_sc as plsc

N_NODES = 10000
N_EDGES = 320000
DN = 128
DE = 16

NC, NS = 2, 16
NW = NC * NS
E_PER_TILE = N_EDGES // NW
ROW = 80
CH_ROWS = 5
CH = ROW * CH_ROWS
N_CH = E_PER_TILE // CH
MROW = 40
M_CH = E_PER_TILE // MROW
M_PAIRS = M_CH // 2
N_PAD = 10240
NODES_PER_TILE = N_PAD // NS
ZR = 64

_mesh = plsc.VectorSubcoreMesh(core_axis_name="c", subcore_axis_name="s")


@functools.partial(
    pl.kernel,
    out_type=jax.ShapeDtypeStruct((NC, N_PAD, DN), jnp.float32),
    mesh=_mesh,
    scratch_types=[
        [pltpu.VMEM((MROW,), jnp.int32)] * 2,
        [pltpu.VMEM((MROW,), jnp.int32)] * 2,
        [pltpu.VMEM((MROW,), jnp.int32)] * 2,
        [pltpu.VMEM((MROW, DN), jnp.float32)] * 2,
        [pltpu.VMEM((MROW, DN), jnp.float32)] * 2,
        [pltpu.VMEM((MROW, DN), jnp.float32)] * 2,
        pltpu.VMEM_SHARED((N_PAD, DN), jnp.float32),
        pltpu.SemaphoreType.DMA,
        pltpu.SemaphoreType.DMA,
        pltpu.SemaphoreType.DMA,
    ],
)
def _msgpass(src_hbm, dst_hbm, ea_hbm, x_hbm, out_hbm,
             idx_s, idx_d, sidx, xg, ea, msg, aggr_ref, sem_i, sem_d, sem_s):
    c = lax.axis_index("c")
    s = lax.axis_index("s")
    wid = c * NS + s
    e0 = wid * E_PER_TILE
    zvec = jnp.zeros((16,), jnp.float32)
    zivec = jnp.zeros((16,), jnp.int32)

    def _esl(k):
        return pl.ds(e0 + k * MROW, MROW)

    def _zero_fill(buf):
        def _zrow(i, _):
            for j in range(DN // 16):
                buf[i, pl.ds(j * 16, 16)] = zvec
            return 0
        lax.fori_loop(0, MROW, _zrow, 0)

    _zero_fill(msg[0])
    _zero_fill(msg[1])
    my_node0 = s * NODES_PER_TILE
    for k in range(NODES_PER_TILE // MROW):
        pltpu.sync_copy(msg[0].at[pl.ds(0, MROW)],
                        aggr_ref.at[pl.ds(my_node0 + k * MROW, MROW)])
    for b in range(2):
        for off in (0, 16, 24):
            sidx[b][pl.ds(off, 16)] = zivec
    plsc.subcore_barrier()

    for b in range(2):
        pltpu.async_copy(msg[b], aggr_ref.at[sidx[b]], sem_s, add=True)

    pltpu.sync_copy(src_hbm.at[_esl(0)], idx_s[0])
    pltpu.sync_copy(dst_hbm.at[_esl(0)], idx_d[0])
    pltpu.async_copy(ea_hbm.at[_esl(0)], ea[0], sem_d)
    pltpu.async_copy(x_hbm.at[idx_s[0]], xg[0], sem_d)
    pltpu.async_copy(src_hbm.at[_esl(1)], idx_s[1], sem_i)
    pltpu.async_copy(dst_hbm.at[_esl(1)], idx_d[1], sem_i)

    def _body(j, b):
        nb = 1 - b
        jn = jnp.minimum(j + 1, M_CH - 1)
        jnn = jnp.minimum(j + 2, M_CH - 1)
        pltpu.make_async_copy(src_hbm.at[_esl(jn)], idx_s[nb], sem_i).wait()
        pltpu.make_async_copy(dst_hbm.at[_esl(jn)], idx_d[nb], sem_i).wait()
        pltpu.async_copy(ea_hbm.at[_esl(jn)], ea[nb], sem_d)
        pltpu.async_copy(x_hbm.at[idx_s[nb]], xg[nb], sem_d)
        pltpu.make_async_copy(msg[b], aggr_ref.at[sidx[b]], sem_s).wait()
        pltpu.make_async_copy(ea_hbm.at[_esl(j)], ea[b], sem_d).wait()
        pltpu.make_async_copy(x_hbm.at[idx_s[b]], xg[b], sem_d).wait()

        def _edge(e, _):
            for jj in range(DN // 16):
                sl = pl.ds(jj * 16, 16)
                msg[b][e, sl] = jnp.maximum(ea[b][e, sl] + xg[b][e, sl], 0.0)
            return 0
        lax.fori_loop(0, MROW, _edge, 0)

        for off in (0, 16, 24):
            sl = pl.ds(off, 16)
            sidx[b][sl] = idx_d[b][sl]
        pltpu.async_copy(src_hbm.at[_esl(jnn)], idx_s[b], sem_i)
        pltpu.async_copy(dst_hbm.at[_esl(jnn)], idx_d[b], sem_i)
        pltpu.async_copy(msg[b], aggr_ref.at[sidx[b]], sem_s, add=True)

    def _pair(g, _):
        _body(2 * g, 0)
        _body(2 * g + 1, 1)
        return 0

    lax.fori_loop(0, M_PAIRS, _pair, 0)

    pltpu.make_async_copy(src_hbm.at[_esl(0)], idx_s[1], sem_i).wait()
    pltpu.make_async_copy(dst_hbm.at[_esl(0)], idx_d[1], sem_i).wait()
    pltpu.make_async_copy(ea_hbm.at[_esl(0)], ea[0], sem_d).wait()
    pltpu.make_async_copy(x_hbm.at[idx_s[0]], xg[0], sem_d).wait()
    pltpu.make_async_copy(msg[0], aggr_ref.at[sidx[0]], sem_s).wait()
    pltpu.make_async_copy(msg[1], aggr_ref.at[sidx[1]], sem_s).wait()
    plsc.subcore_barrier()

    for k in range(NODES_PER_TILE // MROW):
        r = my_node0 + k * MROW
        pltpu.sync_copy(aggr_ref.at[pl.ds(r, MROW)], msg[0].at[pl.ds(0, MROW)])
        pltpu.sync_copy(msg[0].at[pl.ds(0, MROW)], out_hbm.at[c, pl.ds(r, MROW)])


@functools.partial(
    pl.kernel,
    out_type=jax.ShapeDtypeStruct((N_EDGES, DE), jnp.float32),
    mesh=_mesh,
    scratch_types=[
        pltpu.VMEM((CH,), jnp.int32),
        pltpu.VMEM((CH,), jnp.int32),
        pltpu.VMEM((CH, DE), jnp.float32),
        pltpu.VMEM((CH, DE), jnp.float32),
        pltpu.VMEM((CH, DE), jnp.float32),
        pltpu.SemaphoreType.DMA,
    ],
    compiler_params=pltpu.CompilerParams(use_tc_tiling_on_sc=False),
)
def _edgegather(src_hbm, dst_hbm, p_hbm, q_hbm, r_hbm, t_hbm,
                idx_s, idx_d, pg, qg, rb, sem):
    c = lax.axis_index("c")
    s = lax.axis_index("s")
    wid = c * NS + s
    e0 = wid * E_PER_TILE

    def _chunk(k, _):
        pltpu.sync_copy(src_hbm.at[pl.ds(e0 + k * CH, CH)], idx_s)
        pltpu.sync_copy(dst_hbm.at[pl.ds(e0 + k * CH, CH)], idx_d)
        pltpu.sync_copy(r_hbm.at[pl.ds(e0 + k * CH, CH)], rb)
        cps = []
        for j in range(CH_ROWS):
            sl = pl.ds(j * ROW, ROW)
            cps.append(pltpu.async_copy(p_hbm.at[idx_s.at[sl]],
                                        pg.at[sl], sem))
            cps.append(pltpu.async_copy(q_hbm.at[idx_d.at[sl]],
                                        qg.at[sl], sem))
        for cp in cps:
            cp.wait()

        def _edge(e, _):
            rb[e] = jnp.maximum(rb[e] + pg[e] + qg[e], 0.0)
            return 0

        lax.fori_loop(0, CH, _edge, 0)
        pltpu.sync_copy(rb, t_hbm.at[pl.ds(e0 + k * CH, CH)])
        return 0

    lax.fori_loop(0, N_CH, _chunk, 0)


_EBLK = 8000
_EGRID = N_EDGES // _EBLK
_OBLK = 16000
_OGRID = N_EDGES // _OBLK


def _edense_body(e_ref, w0_ref, b0_ref, w1_ref, b1_ref, wc_ref, bc_ref,
                 ea0_ref, ea1_ref, r_ref):
    e = e_ref[...]
    ea0_ref[...] = jnp.dot(e, w0_ref[...],
                           preferred_element_type=jnp.float32) + b0_ref[...]
    ea1_ref[...] = jnp.dot(e, w1_ref[...],
                           preferred_element_type=jnp.float32) + b1_ref[...]
    r_ref[...] = jnp.dot(e, wc_ref[...],
                         preferred_element_type=jnp.float32) + bc_ref[...]


def _edense(edge_attr, w0, b0, w1, b1, wc, bc):
    full = lambda shp: pl.BlockSpec(shp, lambda i: (0, 0))
    return pl.pallas_call(
        _edense_body,
        grid=(_EGRID,),
        in_specs=[
            pl.BlockSpec((_EBLK, DE), lambda i: (i, 0)),
            full((DE, DN)), full((1, DN)),
            full((DE, DN)), full((1, DN)),
            full((DE, DE)), full((1, DE)),
        ],
        out_specs=[
            pl.BlockSpec((_EBLK, DN), lambda i: (i, 0)),
            pl.BlockSpec((_EBLK, DN), lambda i: (i, 0)),
            pl.BlockSpec((_EBLK, DE), lambda i: (i, 0)),
        ],
        out_shape=[
            jax.ShapeDtypeStruct((N_EDGES, DN), jnp.float32),
            jax.ShapeDtypeStruct((N_EDGES, DN), jnp.float32),
            jax.ShapeDtypeStruct((N_EDGES, DE), jnp.float32),
        ],
    )(edge_attr, w0, b0, w1, b1, wc, bc)


def _node_body(x_ref, a_ref, w1_ref, b1_ref, w2_ref, b2_ref, gb_ref,
               epsb_ref, wp_ref, wq_ref, xn_ref, pp_ref, qq_ref):
    x = x_ref[...]
    aggr = (a_ref[0] + a_ref[1])[:N_NODES]
    h = epsb_ref[...] * x + aggr
    h = jnp.maximum(jnp.dot(h, w1_ref[...],
                            preferred_element_type=jnp.float32) + b1_ref[...],
                    0.0)
    h = jnp.dot(h, w2_ref[...],
                preferred_element_type=jnp.float32) + b2_ref[...]
    mean = jnp.mean(h, axis=0, keepdims=True)
    cent = h - mean
    var = jnp.mean(cent * cent, axis=0, keepdims=True)
    bn = gb_ref[0:1, :] * cent * lax.rsqrt(var + 1e-5) + gb_ref[1:2, :]
    xn = (x + jnp.maximum(bn, 0.0)) * 0.5
    xn_ref[...] = xn
    pp_ref[...] = jnp.dot(xn, wp_ref[...], preferred_element_type=jnp.float32)
    qq_ref[...] = jnp.dot(xn, wq_ref[...], preferred_element_type=jnp.float32)


def _node(x, a2, w1, b1, w2, b2, gb, epsb, wp, wq):
    return pl.pallas_call(
        _node_body,
        out_shape=[
            jax.ShapeDtypeStruct((N_NODES, DN), jnp.float32),
            jax.ShapeDtypeStruct((N_NODES, DE), jnp.float32),
            jax.ShapeDtypeStruct((N_NODES, DE), jnp.float32),
        ],
    )(x, a2, w1, b1, w2, b2, gb, epsb, wp, wq)


def _eout_body(t_ref, e_ref, w_ref, b_ref, o_ref):
    mlp = jnp.dot(t_ref[...], w_ref[...],
                  preferred_element_type=jnp.float32) + b_ref[...]
    o_ref[...] = e_ref[...] + mlp * 0.5


def _eout(t, edge_attr, w, b):
    full = lambda shp: pl.BlockSpec(shp, lambda i: (0, 0))
    return pl.pallas_call(
        _eout_body,
        grid=(_OGRID,),
        in_specs=[
            pl.BlockSpec((_OBLK, DE), lambda i: (i, 0)),
            pl.BlockSpec((_OBLK, DE), lambda i: (i, 0)),
            full((DE, DE)), full((1, DE)),
        ],
        out_specs=pl.BlockSpec((_OBLK, DE), lambda i: (i, 0)),
        out_shape=jax.ShapeDtypeStruct((N_EDGES, DE), jnp.float32),
    )(t, edge_attr, w, b)


def kernel(x, edge_index, edge_attr, params):
    src_i = edge_index[0].astype(jnp.int32)
    dst_i = edge_index[1].astype(jnp.int32)
    p0, p1 = params[0], params[1]

    wp = p1['Wm1'][0:DN]
    wq = p1['Wm1'][DN:2 * DN]
    wc = p1['Wm1'][2 * DN:]

    ea0, ea1, rmat = _edense(
        edge_attr,
        p0['We'], p0['be'].reshape(1, DN),
        p1['We'], p1['be'].reshape(1, DN),
        wc, p1['bm1'].reshape(1, DE),
    )

    ones = jnp.ones((1, DN), jnp.float32)
    for p, ea in ((p0, ea0), (p1, ea1)):
        a2 = _msgpass(src_i, dst_i, ea, x)
        gb = jnp.stack([p['bn_gamma'], p['bn_beta']])
        epsb = (1.0 + p['eps']) * ones
        x, pp, qq = _node(x, a2, p['W1'], p['b1'].reshape(1, DN),
                          p['W2'], p['b2'].reshape(1, DN), gb, epsb, wp, wq)

    t = _edgegather(src_i, dst_i, pp, qq, rmat)
    e_out = _eout(t, edge_attr, p1['Wm2'], p1['bm2'].reshape(1, DE))
    return (x, e_out)


# speedup vs baseline: 3.6486x; 1.0587x over previous
import functools

import jax
import jax.numpy as jnp
from jax import lax
from jax.experimental import pallas as pl
from jax.experimental.pallas import tpu as pltpu
from jax.experimental.pallas import tpu_sc as plsc

N_NODES = 10000
N_EDGES = 320000
DN = 128
DE = 16

NC, NS = 2, 16
NW = NC * NS
E_PER_TILE = N_EDGES // NW
ROW = 80
CH_ROWS = 5
CH = ROW * CH_ROWS
N_CH = E_PER_TILE // CH
MROW = 40
M_CH = E_PER_TILE // MROW
M_PAIRS = M_CH // 2
N_PAD = 10240
NODES_PER_TILE = N_PAD // NS
ZR = 64

_mesh = plsc.VectorSubcoreMesh(core_axis_name="c", subcore_axis_name="s")


@functools.partial(
    pl.kernel,
    out_type=jax.ShapeDtypeStruct((NC, N_PAD, DN), jnp.float32),
    mesh=_mesh,
    scratch_types=[
        [pltpu.VMEM((MROW,), jnp.int32)] * 2,
        [pltpu.VMEM((MROW,), jnp.int32)] * 2,
        [pltpu.VMEM((MROW,), jnp.int32)] * 2,
        [pltpu.VMEM((MROW, DN), jnp.float32)] * 2,
        [pltpu.VMEM((MROW, DN), jnp.float32)] * 2,
        [pltpu.VMEM((MROW, DN), jnp.float32)] * 2,
        pltpu.VMEM_SHARED((N_PAD, DN), jnp.float32),
        pltpu.SemaphoreType.DMA,
        pltpu.SemaphoreType.DMA,
        pltpu.SemaphoreType.DMA,
    ],
)
def _msgpass(src_hbm, dst_hbm, ea_hbm, x_hbm, out_hbm,
             idx_s, idx_d, sidx, xg, ea, msg, aggr_ref, sem_i, sem_d, sem_s):
    c = lax.axis_index("c")
    s = lax.axis_index("s")
    wid = c * NS + s
    e0 = wid * E_PER_TILE
    zvec = jnp.zeros((16,), jnp.float32)
    zivec = jnp.full((16,), N_PAD - 8, jnp.int32)

    def _esl(k):
        return pl.ds(e0 + k * MROW, MROW)

    def _zero_fill(buf):
        def _zrow(i, _):
            for j in range(DN // 16):
                buf[i, pl.ds(j * 16, 16)] = zvec
            return 0
        lax.fori_loop(0, MROW, _zrow, 0)

    _zero_fill(msg[0])
    _zero_fill(msg[1])
    my_node0 = s * NODES_PER_TILE
    for k in range(NODES_PER_TILE // MROW):
        pltpu.sync_copy(msg[0].at[pl.ds(0, MROW)],
                        aggr_ref.at[pl.ds(my_node0 + k * MROW, MROW)])
    for b in range(2):
        for off in (0, 16, 24):
            sidx[b][pl.ds(off, 16)] = zivec
    plsc.subcore_barrier()

    for b in range(2):
        pltpu.async_copy(msg[b], aggr_ref.at[sidx[b]], sem_s, add=True)

    pltpu.sync_copy(src_hbm.at[_esl(0)], idx_s[0])
    pltpu.sync_copy(dst_hbm.at[_esl(0)], idx_d[0])
    pltpu.async_copy(ea_hbm.at[_esl(0)], ea[0], sem_d)
    pltpu.async_copy(x_hbm.at[idx_s[0]], xg[0], sem_d)
    pltpu.async_copy(src_hbm.at[_esl(1)], idx_s[1], sem_i)
    pltpu.async_copy(dst_hbm.at[_esl(1)], idx_d[1], sem_i)

    def _body(j, b):
        nb = 1 - b
        jn = jnp.minimum(j + 1, M_CH - 1)
        jnn = jnp.minimum(j + 2, M_CH - 1)
        pltpu.make_async_copy(src_hbm.at[_esl(jn)], idx_s[nb], sem_i).wait()
        pltpu.make_async_copy(dst_hbm.at[_esl(jn)], idx_d[nb], sem_i).wait()
        pltpu.async_copy(ea_hbm.at[_esl(jn)], ea[nb], sem_d)
        pltpu.async_copy(x_hbm.at[idx_s[nb]], xg[nb], sem_d)
        pltpu.make_async_copy(msg[b], aggr_ref.at[sidx[b]], sem_s).wait()
        pltpu.make_async_copy(ea_hbm.at[_esl(j)], ea[b], sem_d).wait()
        pltpu.make_async_copy(x_hbm.at[idx_s[b]], xg[b], sem_d).wait()

        def _edge(e, _):
            for jj in range(DN // 16):
                sl = pl.ds(jj * 16, 16)
                msg[b][e, sl] = jnp.maximum(ea[b][e, sl] + xg[b][e, sl], 0.0)
            return 0
        lax.fori_loop(0, MROW, _edge, 0)

        for off in (0, 16, 24):
            sl = pl.ds(off, 16)
            sidx[b][sl] = idx_d[b][sl]
        pltpu.async_copy(src_hbm.at[_esl(jnn)], idx_s[b], sem_i)
        pltpu.async_copy(dst_hbm.at[_esl(jnn)], idx_d[b], sem_i)
        pltpu.async_copy(msg[b], aggr_ref.at[sidx[b]], sem_s, add=True)

    def _pair(g, _):
        _body(2 * g, 0)
        _body(2 * g + 1, 1)
        return 0

    lax.fori_loop(0, M_PAIRS, _pair, 0)

    pltpu.make_async_copy(src_hbm.at[_esl(0)], idx_s[1], sem_i).wait()
    pltpu.make_async_copy(dst_hbm.at[_esl(0)], idx_d[1], sem_i).wait()
    pltpu.make_async_copy(ea_hbm.at[_esl(0)], ea[0], sem_d).wait()
    pltpu.make_async_copy(x_hbm.at[idx_s[0]], xg[0], sem_d).wait()
    pltpu.make_async_copy(msg[0], aggr_ref.at[sidx[0]], sem_s).wait()
    pltpu.make_async_copy(msg[1], aggr_ref.at[sidx[1]], sem_s).wait()
    plsc.subcore_barrier()

    for k in range(NODES_PER_TILE // MROW):
        r = my_node0 + k * MROW
        pltpu.sync_copy(aggr_ref.at[pl.ds(r, MROW)], msg[0].at[pl.ds(0, MROW)])
        pltpu.sync_copy(msg[0].at[pl.ds(0, MROW)], out_hbm.at[c, pl.ds(r, MROW)])


@functools.partial(
    pl.kernel,
    out_type=jax.ShapeDtypeStruct((N_EDGES, DE), jnp.float32),
    mesh=_mesh,
    scratch_types=[
        [pltpu.VMEM((CH,), jnp.int32)] * 2,
        [pltpu.VMEM((CH,), jnp.int32)] * 2,
        [pltpu.VMEM((CH, DE), jnp.float32)] * 2,
        [pltpu.VMEM((CH, DE), jnp.float32)] * 2,
        [pltpu.VMEM((CH, DE), jnp.float32)] * 2,
        pltpu.SemaphoreType.DMA,
        pltpu.SemaphoreType.DMA,
        pltpu.SemaphoreType.DMA,
    ],
    compiler_params=pltpu.CompilerParams(use_tc_tiling_on_sc=False),
)
def _edgegather(src_hbm, dst_hbm, p_hbm, q_hbm, r_hbm, t_hbm,
                idx_s, idx_d, pg, qg, rb, sem_i, sem_d, sem_o):
    c = lax.axis_index("c")
    s = lax.axis_index("s")
    wid = c * NS + s
    e0 = wid * E_PER_TILE

    def _esl(k):
        return pl.ds(e0 + k * CH, CH)

    def _start_data(k, b):
        pltpu.async_copy(r_hbm.at[_esl(k)], rb[b], sem_d)
        for j in range(CH_ROWS):
            sl = pl.ds(j * ROW, ROW)
            pltpu.async_copy(p_hbm.at[idx_s[b].at[sl]], pg[b].at[sl], sem_d)
            pltpu.async_copy(q_hbm.at[idx_d[b].at[sl]], qg[b].at[sl], sem_d)

    def _wait_data(k, b):
        pltpu.make_async_copy(r_hbm.at[_esl(k)], rb[b], sem_d).wait()
        for j in range(CH_ROWS):
            sl = pl.ds(j * ROW, ROW)
            pltpu.make_async_copy(p_hbm.at[idx_s[b].at[sl]],
                                  pg[b].at[sl], sem_d).wait()
            pltpu.make_async_copy(q_hbm.at[idx_d[b].at[sl]],
                                  qg[b].at[sl], sem_d).wait()

    def _body(j, b, first=False):
        nb = 1 - b
        jn = jnp.minimum(j + 1, N_CH - 1)
        jnn = jnp.minimum(j + 2, N_CH - 1)
        pltpu.make_async_copy(src_hbm.at[_esl(jn)], idx_s[nb], sem_i).wait()
        pltpu.make_async_copy(dst_hbm.at[_esl(jn)], idx_d[nb], sem_i).wait()
        if not first:
            pltpu.make_async_copy(rb[nb], t_hbm.at[_esl(j)], sem_o).wait()
        _start_data(jn, nb)
        _wait_data(j, b)

        def _edge(e, _):
            rb[b][e] = jnp.maximum(rb[b][e] + pg[b][e] + qg[b][e], 0.0)
            return 0

        lax.fori_loop(0, CH, _edge, 0)
        pltpu.async_copy(src_hbm.at[_esl(jnn)], idx_s[b], sem_i)
        pltpu.async_copy(dst_hbm.at[_esl(jnn)], idx_d[b], sem_i)
        pltpu.async_copy(rb[b], t_hbm.at[_esl(j)], sem_o)

    pltpu.sync_copy(src_hbm.at[_esl(0)], idx_s[0])
    pltpu.sync_copy(dst_hbm.at[_esl(0)], idx_d[0])
    _start_data(0, 0)
    pltpu.async_copy(src_hbm.at[_esl(1)], idx_s[1], sem_i)
    pltpu.async_copy(dst_hbm.at[_esl(1)], idx_d[1], sem_i)

    _body(0, 0, first=True)

    def _pair(g, _):
        _body(2 * g + 1, 1)
        _body(2 * g + 2, 0)
        return 0

    lax.fori_loop(0, (N_CH - 1) // 2, _pair, 0)

    pltpu.make_async_copy(src_hbm.at[_esl(0)], idx_s[0], sem_i).wait()
    pltpu.make_async_copy(dst_hbm.at[_esl(0)], idx_d[0], sem_i).wait()
    _wait_data(N_CH - 1, 1)
    pltpu.make_async_copy(rb[0], t_hbm.at[_esl(N_CH - 1)], sem_o).wait()


_EBLK = 8000
_EGRID = N_EDGES // _EBLK
_OBLK = 16000
_OGRID = N_EDGES // _OBLK


def _ea_body(e_ref, w0_ref, b0_ref, ea0_ref):
    ea0_ref[...] = jnp.dot(e_ref[...], w0_ref[...],
                           preferred_element_type=jnp.float32) + b0_ref[...]


def _ea_proj(edge_attr, w0, b0):
    full = lambda shp: pl.BlockSpec(shp, lambda i: (0, 0))
    return pl.pallas_call(
        _ea_body,
        grid=(_EGRID,),
        in_specs=[
            pl.BlockSpec((_EBLK, DE), lambda i: (i, 0)),
            full((DE, DN)), full((1, DN)),
        ],
        out_specs=pl.BlockSpec((_EBLK, DN), lambda i: (i, 0)),
        out_shape=jax.ShapeDtypeStruct((N_EDGES, DN), jnp.float32),
    )(edge_attr, w0, b0)


def _ea_r_body(e_ref, w1_ref, b1_ref, wc_ref, bc_ref, ea1_ref, r_ref):
    e = e_ref[...]
    ea1_ref[...] = jnp.dot(e, w1_ref[...],
                           preferred_element_type=jnp.float32) + b1_ref[...]
    r_ref[...] = jnp.dot(e, wc_ref[...],
                         preferred_element_type=jnp.float32) + bc_ref[...]


def _ea_r_proj(edge_attr, w1, b1, wc, bc):
    full = lambda shp: pl.BlockSpec(shp, lambda i: (0, 0))
    return pl.pallas_call(
        _ea_r_body,
        grid=(_EGRID,),
        in_specs=[
            pl.BlockSpec((_EBLK, DE), lambda i: (i, 0)),
            full((DE, DN)), full((1, DN)),
            full((DE, DE)), full((1, DE)),
        ],
        out_specs=[
            pl.BlockSpec((_EBLK, DN), lambda i: (i, 0)),
            pl.BlockSpec((_EBLK, DE), lambda i: (i, 0)),
        ],
        out_shape=[
            jax.ShapeDtypeStruct((N_EDGES, DN), jnp.float32),
            jax.ShapeDtypeStruct((N_EDGES, DE), jnp.float32),
        ],
    )(edge_attr, w1, b1, wc, bc)


def _node_body(x_ref, a_ref, w1_ref, b1_ref, w2_ref, b2_ref, gb_ref,
               epsb_ref, wp_ref, wq_ref, xn_ref, pp_ref, qq_ref):
    x = x_ref[...]
    aggr = (a_ref[0] + a_ref[1])[:N_NODES]
    h = epsb_ref[...] * x + aggr
    h = jnp.maximum(jnp.dot(h, w1_ref[...],
                            preferred_element_type=jnp.float32) + b1_ref[...],
                    0.0)
    h = jnp.dot(h, w2_ref[...],
                preferred_element_type=jnp.float32) + b2_ref[...]
    mean = jnp.mean(h, axis=0, keepdims=True)
    cent = h - mean
    var = jnp.mean(cent * cent, axis=0, keepdims=True)
    bn = gb_ref[0:1, :] * cent * lax.rsqrt(var + 1e-5) + gb_ref[1:2, :]
    xn = (x + jnp.maximum(bn, 0.0)) * 0.5
    xn_ref[...] = xn
    pp_ref[...] = jnp.dot(xn, wp_ref[...], preferred_element_type=jnp.float32)
    qq_ref[...] = jnp.dot(xn, wq_ref[...], preferred_element_type=jnp.float32)


def _node(x, a2, w1, b1, w2, b2, gb, epsb, wp, wq):
    return pl.pallas_call(
        _node_body,
        out_shape=[
            jax.ShapeDtypeStruct((N_NODES, DN), jnp.float32),
            jax.ShapeDtypeStruct((N_NODES, DE), jnp.float32),
            jax.ShapeDtypeStruct((N_NODES, DE), jnp.float32),
        ],
    )(x, a2, w1, b1, w2, b2, gb, epsb, wp, wq)


def _eout_body(t_ref, e_ref, w_ref, b_ref, o_ref):
    mlp = jnp.dot(t_ref[...], w_ref[...],
                  preferred_element_type=jnp.float32) + b_ref[...]
    o_ref[...] = e_ref[...] + mlp * 0.5


def _eout(t, edge_attr, w, b):
    full = lambda shp: pl.BlockSpec(shp, lambda i: (0, 0))
    return pl.pallas_call(
        _eout_body,
        grid=(_OGRID,),
        in_specs=[
            pl.BlockSpec((_OBLK, DE), lambda i: (i, 0)),
            pl.BlockSpec((_OBLK, DE), lambda i: (i, 0)),
            full((DE, DE)), full((1, DE)),
        ],
        out_specs=pl.BlockSpec((_OBLK, DE), lambda i: (i, 0)),
        out_shape=jax.ShapeDtypeStruct((N_EDGES, DE), jnp.float32),
    )(t, edge_attr, w, b)


def kernel(x, edge_index, edge_attr, params):
    src_i = edge_index[0].astype(jnp.int32)
    dst_i = edge_index[1].astype(jnp.int32)
    p0, p1 = params[0], params[1]

    wp = p1['Wm1'][0:DN]
    wq = p1['Wm1'][DN:2 * DN]
    wc = p1['Wm1'][2 * DN:]

    ea0 = _ea_proj(edge_attr, p0['We'], p0['be'].reshape(1, DN))
    ea1, rmat = _ea_r_proj(edge_attr, p1['We'], p1['be'].reshape(1, DN),
                           wc, p1['bm1'].reshape(1, DE))

    ones = jnp.ones((1, DN), jnp.float32)
    for p, ea in ((p0, ea0), (p1, ea1)):
        a2 = _msgpass(src_i, dst_i, ea, x)
        gb = jnp.stack([p['bn_gamma'], p['bn_beta']])
        epsb = (1.0 + p['eps']) * ones
        x, pp, qq = _node(x, a2, p['W1'], p['b1'].reshape(1, DN),
                          p['W2'], p['b2'].reshape(1, DN), gb, epsb, wp, wq)

    t = _edgegather(src_i, dst_i, pp, qq, rmat)
    e_out = _eout(t, edge_attr, p1['Wm2'], p1['bm2'].reshape(1, DE))
    return (x, e_out)


# speedup vs baseline: 3.7321x; 1.0229x over previous
import functools

import jax
import jax.numpy as jnp
from jax import lax
from jax.experimental import pallas as pl
from jax.experimental.pallas import tpu as pltpu
from jax.experimental.pallas import tpu_sc as plsc

N_NODES = 10000
N_EDGES = 320000
DN = 128
DE = 16

NC, NS = 2, 16
NW = NC * NS
E_PER_TILE = N_EDGES // NW
ROW = 80
CH_ROWS = 5
CH = ROW * CH_ROWS
N_CH = E_PER_TILE // CH
MROW = 40
M_CH = E_PER_TILE // MROW
M_PAIRS = M_CH // 2
N_PAD = 10240
NODES_PER_TILE = N_PAD // NS
ZR = 64

_mesh = plsc.VectorSubcoreMesh(core_axis_name="c", subcore_axis_name="s")


@functools.partial(
    pl.kernel,
    out_type=jax.ShapeDtypeStruct((NC, N_PAD, DN), jnp.float32),
    mesh=_mesh,
    scratch_types=[
        [pltpu.VMEM((MROW,), jnp.int32)] * 2,
        [pltpu.VMEM((MROW,), jnp.int32)] * 2,
        [pltpu.VMEM((MROW,), jnp.int32)] * 2,
        [pltpu.VMEM((MROW, DN), jnp.float32)] * 2,
        [pltpu.VMEM((MROW, DN), jnp.float32)] * 2,
        [pltpu.VMEM((MROW, DN), jnp.float32)] * 2,
        pltpu.VMEM_SHARED((N_PAD, DN), jnp.float32),
        pltpu.SemaphoreType.DMA,
        pltpu.SemaphoreType.DMA,
        pltpu.SemaphoreType.DMA,
    ],
)
def _msgpass(src_hbm, dst_hbm, ea_hbm, x_hbm, out_hbm,
             idx_s, idx_d, sidx, xg, ea, msg, aggr_ref, sem_i, sem_d, sem_s):
    c = lax.axis_index("c")
    s = lax.axis_index("s")
    wid = c * NS + s
    e0 = wid * E_PER_TILE
    zvec = jnp.zeros((16,), jnp.float32)
    zivec = jnp.full((16,), N_PAD - 8, jnp.int32)

    def _esl(k):
        return pl.ds(e0 + k * MROW, MROW)

    def _zero_fill(buf):
        def _zrow(i, _):
            for j in range(DN // 16):
                buf[i, pl.ds(j * 16, 16)] = zvec
            return 0
        lax.fori_loop(0, MROW, _zrow, 0)

    _zero_fill(msg[0])
    _zero_fill(msg[1])
    my_node0 = s * NODES_PER_TILE
    for k in range(NODES_PER_TILE // MROW):
        pltpu.sync_copy(msg[0].at[pl.ds(0, MROW)],
                        aggr_ref.at[pl.ds(my_node0 + k * MROW, MROW)])
    for b in range(2):
        for off in (0, 16, 24):
            sidx[b][pl.ds(off, 16)] = zivec
    plsc.subcore_barrier()

    for b in range(2):
        pltpu.async_copy(msg[b], aggr_ref.at[sidx[b]], sem_s, add=True)

    pltpu.sync_copy(src_hbm.at[_esl(0)], idx_s[0])
    pltpu.sync_copy(dst_hbm.at[_esl(0)], idx_d[0])
    pltpu.async_copy(ea_hbm.at[_esl(0)], ea[0], sem_d)
    pltpu.async_copy(x_hbm.at[idx_s[0]], xg[0], sem_d)
    pltpu.async_copy(src_hbm.at[_esl(1)], idx_s[1], sem_i)
    pltpu.async_copy(dst_hbm.at[_esl(1)], idx_d[1], sem_i)

    def _body(j, b):
        nb = 1 - b
        jn = jnp.minimum(j + 1, M_CH - 1)
        jnn = jnp.minimum(j + 2, M_CH - 1)
        pltpu.make_async_copy(src_hbm.at[_esl(jn)], idx_s[nb], sem_i).wait()
        pltpu.make_async_copy(dst_hbm.at[_esl(jn)], idx_d[nb], sem_i).wait()
        pltpu.async_copy(ea_hbm.at[_esl(jn)], ea[nb], sem_d)
        pltpu.async_copy(x_hbm.at[idx_s[nb]], xg[nb], sem_d)
        pltpu.make_async_copy(msg[b], aggr_ref.at[sidx[b]], sem_s).wait()
        pltpu.make_async_copy(ea_hbm.at[_esl(j)], ea[b], sem_d).wait()
        pltpu.make_async_copy(x_hbm.at[idx_s[b]], xg[b], sem_d).wait()

        def _edge(e, _):
            for jj in range(DN // 16):
                sl = pl.ds(jj * 16, 16)
                msg[b][e, sl] = jnp.maximum(ea[b][e, sl] + xg[b][e, sl], 0.0)
            return 0
        lax.fori_loop(0, MROW, _edge, 0)

        for off in (0, 16, 24):
            sl = pl.ds(off, 16)
            sidx[b][sl] = idx_d[b][sl]
        pltpu.async_copy(src_hbm.at[_esl(jnn)], idx_s[b], sem_i)
        pltpu.async_copy(dst_hbm.at[_esl(jnn)], idx_d[b], sem_i)
        pltpu.async_copy(msg[b], aggr_ref.at[sidx[b]], sem_s, add=True)

    def _pair(g, _):
        _body(2 * g, 0)
        _body(2 * g + 1, 1)
        return 0

    lax.fori_loop(0, M_PAIRS, _pair, 0)

    pltpu.make_async_copy(src_hbm.at[_esl(0)], idx_s[1], sem_i).wait()
    pltpu.make_async_copy(dst_hbm.at[_esl(0)], idx_d[1], sem_i).wait()
    pltpu.make_async_copy(ea_hbm.at[_esl(0)], ea[0], sem_d).wait()
    pltpu.make_async_copy(x_hbm.at[idx_s[0]], xg[0], sem_d).wait()
    pltpu.make_async_copy(msg[0], aggr_ref.at[sidx[0]], sem_s).wait()
    pltpu.make_async_copy(msg[1], aggr_ref.at[sidx[1]], sem_s).wait()
    plsc.subcore_barrier()

    for k in range(NODES_PER_TILE // MROW):
        r = my_node0 + k * MROW
        pltpu.sync_copy(aggr_ref.at[pl.ds(r, MROW)], msg[0].at[pl.ds(0, MROW)])
        pltpu.sync_copy(msg[0].at[pl.ds(0, MROW)], out_hbm.at[c, pl.ds(r, MROW)])


@functools.partial(
    pl.kernel,
    out_type=jax.ShapeDtypeStruct((N_EDGES * DE,), jnp.float32),
    mesh=_mesh,
    scratch_types=[
        [pltpu.VMEM((CH,), jnp.int32)] * 2,
        [pltpu.VMEM((CH,), jnp.int32)] * 2,
        [pltpu.VMEM((CH, DE), jnp.float32)] * 2,
        [pltpu.VMEM((CH, DE), jnp.float32)] * 2,
        [pltpu.VMEM((CH, DE), jnp.float32)] * 2,
        [pltpu.VMEM((CH * DE,), jnp.float32)] * 2,
        pltpu.SemaphoreType.DMA,
        pltpu.SemaphoreType.DMA,
        pltpu.SemaphoreType.DMA,
    ],
    compiler_params=pltpu.CompilerParams(use_tc_tiling_on_sc=False),
)
def _edgegather(src_hbm, dst_hbm, p_hbm, q_hbm, r_hbm, t_hbm,
                idx_s, idx_d, pg, qg, rr, rb, sem_i, sem_d, sem_o):
    c = lax.axis_index("c")
    s = lax.axis_index("s")
    wid = c * NS + s
    e0 = wid * E_PER_TILE

    def _esl(k):
        return pl.ds(e0 + k * CH, CH)

    def _tsl(k):
        return pl.ds((e0 + k * CH) * DE, CH * DE)

    def _start_data(k, b):
        pltpu.async_copy(r_hbm.at[_esl(k)], rr[b], sem_d)
        for j in range(CH_ROWS):
            sl = pl.ds(j * ROW, ROW)
            pltpu.async_copy(p_hbm.at[idx_s[b].at[sl]], pg[b].at[sl], sem_d)
            pltpu.async_copy(q_hbm.at[idx_d[b].at[sl]], qg[b].at[sl], sem_d)

    def _wait_data(k, b):
        pltpu.make_async_copy(r_hbm.at[_esl(k)], rr[b], sem_d).wait()
        for j in range(CH_ROWS):
            sl = pl.ds(j * ROW, ROW)
            pltpu.make_async_copy(p_hbm.at[idx_s[b].at[sl]],
                                  pg[b].at[sl], sem_d).wait()
            pltpu.make_async_copy(q_hbm.at[idx_d[b].at[sl]],
                                  qg[b].at[sl], sem_d).wait()

    def _body(j, b, first=False):
        nb = 1 - b
        jn = jnp.minimum(j + 1, N_CH - 1)
        jnn = jnp.minimum(j + 2, N_CH - 1)
        pltpu.make_async_copy(src_hbm.at[_esl(jn)], idx_s[nb], sem_i).wait()
        pltpu.make_async_copy(dst_hbm.at[_esl(jn)], idx_d[nb], sem_i).wait()
        if not first:
            pltpu.make_async_copy(rb[nb], t_hbm.at[_tsl(j)], sem_o).wait()
        _start_data(jn, nb)
        _wait_data(j, b)

        def _edge(e, _):
            rb[b][pl.ds(e * DE, DE)] = jnp.maximum(
                rr[b][e] + pg[b][e] + qg[b][e], 0.0)
            return 0

        lax.fori_loop(0, CH, _edge, 0)
        pltpu.async_copy(src_hbm.at[_esl(jnn)], idx_s[b], sem_i)
        pltpu.async_copy(dst_hbm.at[_esl(jnn)], idx_d[b], sem_i)
        pltpu.async_copy(rb[b], t_hbm.at[_tsl(j)], sem_o)

    pltpu.sync_copy(src_hbm.at[_esl(0)], idx_s[0])
    pltpu.sync_copy(dst_hbm.at[_esl(0)], idx_d[0])
    _start_data(0, 0)
    pltpu.async_copy(src_hbm.at[_esl(1)], idx_s[1], sem_i)
    pltpu.async_copy(dst_hbm.at[_esl(1)], idx_d[1], sem_i)

    _body(0, 0, first=True)

    def _pair(g, _):
        _body(2 * g + 1, 1)
        _body(2 * g + 2, 0)
        return 0

    lax.fori_loop(0, (N_CH - 1) // 2, _pair, 0)

    pltpu.make_async_copy(src_hbm.at[_esl(0)], idx_s[0], sem_i).wait()
    pltpu.make_async_copy(dst_hbm.at[_esl(0)], idx_d[0], sem_i).wait()
    _wait_data(N_CH - 1, 1)
    pltpu.make_async_copy(rb[0], t_hbm.at[_tsl(N_CH - 1)], sem_o).wait()


_EBLK = 8000
_EGRID = N_EDGES // _EBLK
_OBLK = 16000
_OGRID = N_EDGES // _OBLK


def _ea_body(e_ref, w0_ref, b0_ref, ea0_ref):
    ea0_ref[...] = jnp.dot(e_ref[...], w0_ref[...],
                           preferred_element_type=jnp.float32) + b0_ref[...]


def _ea_proj(edge_attr, w0, b0):
    full = lambda shp: pl.BlockSpec(shp, lambda i: (0, 0))
    return pl.pallas_call(
        _ea_body,
        grid=(_EGRID,),
        in_specs=[
            pl.BlockSpec((_EBLK, DE), lambda i: (i, 0)),
            full((DE, DN)), full((1, DN)),
        ],
        out_specs=pl.BlockSpec((_EBLK, DN), lambda i: (i, 0)),
        out_shape=jax.ShapeDtypeStruct((N_EDGES, DN), jnp.float32),
    )(edge_attr, w0, b0)


def _ea_r_body(e_ref, w1_ref, b1_ref, wc_ref, bc_ref, ea1_ref, r_ref):
    e = e_ref[...]
    ea1_ref[...] = jnp.dot(e, w1_ref[...],
                           preferred_element_type=jnp.float32) + b1_ref[...]
    r_ref[...] = jnp.dot(e, wc_ref[...],
                         preferred_element_type=jnp.float32) + bc_ref[...]


def _ea_r_proj(edge_attr, w1, b1, wc, bc):
    full = lambda shp: pl.BlockSpec(shp, lambda i: (0, 0))
    return pl.pallas_call(
        _ea_r_body,
        grid=(_EGRID,),
        in_specs=[
            pl.BlockSpec((_EBLK, DE), lambda i: (i, 0)),
            full((DE, DN)), full((1, DN)),
            full((DE, DE)), full((1, DE)),
        ],
        out_specs=[
            pl.BlockSpec((_EBLK, DN), lambda i: (i, 0)),
            pl.BlockSpec((_EBLK, DE), lambda i: (i, 0)),
        ],
        out_shape=[
            jax.ShapeDtypeStruct((N_EDGES, DN), jnp.float32),
            jax.ShapeDtypeStruct((N_EDGES, DE), jnp.float32),
        ],
    )(edge_attr, w1, b1, wc, bc)


def _node_body(x_ref, a_ref, w1_ref, b1_ref, w2_ref, b2_ref, gb_ref,
               epsb_ref, wp_ref, wq_ref, xn_ref, pp_ref, qq_ref):
    x = x_ref[...]
    aggr = (a_ref[0] + a_ref[1])[:N_NODES]
    h = epsb_ref[...] * x + aggr
    h = jnp.maximum(jnp.dot(h, w1_ref[...],
                            preferred_element_type=jnp.float32) + b1_ref[...],
                    0.0)
    h = jnp.dot(h, w2_ref[...],
                preferred_element_type=jnp.float32) + b2_ref[...]
    mean = jnp.mean(h, axis=0, keepdims=True)
    cent = h - mean
    var = jnp.mean(cent * cent, axis=0, keepdims=True)
    bn = gb_ref[0:1, :] * cent * lax.rsqrt(var + 1e-5) + gb_ref[1:2, :]
    xn = (x + jnp.maximum(bn, 0.0)) * 0.5
    xn_ref[...] = xn
    pp_ref[...] = jnp.dot(xn, wp_ref[...], preferred_element_type=jnp.float32)
    qq_ref[...] = jnp.dot(xn, wq_ref[...], preferred_element_type=jnp.float32)


def _node(x, a2, w1, b1, w2, b2, gb, epsb, wp, wq):
    return pl.pallas_call(
        _node_body,
        out_shape=[
            jax.ShapeDtypeStruct((N_NODES, DN), jnp.float32),
            jax.ShapeDtypeStruct((N_NODES, DE), jnp.float32),
            jax.ShapeDtypeStruct((N_NODES, DE), jnp.float32),
        ],
    )(x, a2, w1, b1, w2, b2, gb, epsb, wp, wq)


_T8ROWS = N_EDGES * DE // DN
_T8BLK = _T8ROWS // _OGRID


def _eout_body(t_ref, w_ref, b_ref, o_ref):
    o_ref[...] = jnp.dot(t_ref[...], w_ref[...],
                         preferred_element_type=jnp.float32) + b_ref[...]


def _eout(t8, w8, b8):
    full = lambda shp: pl.BlockSpec(shp, lambda i: (0, 0))
    return pl.pallas_call(
        _eout_body,
        grid=(_OGRID,),
        in_specs=[
            pl.BlockSpec((_T8BLK, DN), lambda i: (i, 0)),
            full((DN, DN)), full((1, DN)),
        ],
        out_specs=pl.BlockSpec((_T8BLK, DN), lambda i: (i, 0)),
        out_shape=jax.ShapeDtypeStruct((_T8ROWS, DN), jnp.float32),
    )(t8, w8, b8)


def kernel(x, edge_index, edge_attr, params):
    src_i = edge_index[0].astype(jnp.int32)
    dst_i = edge_index[1].astype(jnp.int32)
    p0, p1 = params[0], params[1]

    wp = p1['Wm1'][0:DN]
    wq = p1['Wm1'][DN:2 * DN]
    wc = p1['Wm1'][2 * DN:]

    ea0 = _ea_proj(edge_attr, p0['We'], p0['be'].reshape(1, DN))
    ea1, rmat = _ea_r_proj(edge_attr, p1['We'], p1['be'].reshape(1, DN),
                           wc, p1['bm1'].reshape(1, DE))

    ones = jnp.ones((1, DN), jnp.float32)
    for p, ea in ((p0, ea0), (p1, ea1)):
        a2 = _msgpass(src_i, dst_i, ea, x)
        gb = jnp.stack([p['bn_gamma'], p['bn_beta']])
        epsb = (1.0 + p['eps']) * ones
        x, pp, qq = _node(x, a2, p['W1'], p['b1'].reshape(1, DN),
                          p['W2'], p['b2'].reshape(1, DN), gb, epsb, wp, wq)

    t8 = _edgegather(src_i, dst_i, pp, qq, rmat).reshape(_T8ROWS, DN)
    w8 = jnp.kron(jnp.eye(8, dtype=jnp.float32), p1['Wm2'])
    b8 = jnp.tile(p1['bm2'], 8).reshape(1, DN)
    mlp8 = _eout(t8, w8, b8)
    e_out = edge_attr + mlp8.reshape(N_EDGES, DE) * 0.5
    return (x, e_out)


# speedup vs baseline: 4.3511x; 1.1659x over previous
import functools

import jax
import jax.numpy as jnp
from jax import lax
from jax.experimental import pallas as pl
from jax.experimental.pallas import tpu as pltpu
from jax.experimental.pallas import tpu_sc as plsc

N_NODES = 10000
N_EDGES = 320000
DN = 128
DE = 16

NC, NS = 2, 16
NW = NC * NS
E_PER_TILE = N_EDGES // NW
ROW = 80
CH_ROWS = 5
CH = ROW * CH_ROWS
N_CH = E_PER_TILE // CH
MROW = 40
M_CH = E_PER_TILE // MROW
M_PAIRS = M_CH // 2
N_PAD = 10240
NODES_PER_TILE = N_PAD // NS
ZR = 64

_mesh = plsc.VectorSubcoreMesh(core_axis_name="c", subcore_axis_name="s")


@functools.partial(
    pl.kernel,
    out_type=jax.ShapeDtypeStruct((NC, N_PAD, DN), jnp.float32),
    mesh=_mesh,
    scratch_types=[
        [pltpu.VMEM((MROW,), jnp.int32)] * 2,
        [pltpu.VMEM((MROW,), jnp.int32)] * 2,
        [pltpu.VMEM((MROW,), jnp.int32)] * 2,
        [pltpu.VMEM((MROW, DN), jnp.float32)] * 2,
        [pltpu.VMEM((MROW, DN), jnp.float32)] * 2,
        [pltpu.VMEM((MROW, DN), jnp.float32)] * 2,
        pltpu.VMEM_SHARED((N_PAD, DN), jnp.float32),
        pltpu.SemaphoreType.DMA,
        pltpu.SemaphoreType.DMA,
        pltpu.SemaphoreType.DMA,
    ],
)
def _msgpass(src_hbm, dst_hbm, ea_hbm, x_hbm, out_hbm,
             idx_s, idx_d, sidx, xg, ea, msg, aggr_ref, sem_i, sem_d, sem_s):
    c = lax.axis_index("c")
    s = lax.axis_index("s")
    wid = c * NS + s
    e0 = wid * E_PER_TILE
    zvec = jnp.zeros((16,), jnp.float32)
    zivec = jnp.full((16,), N_PAD - 8, jnp.int32)

    def _esl(k):
        return pl.ds(e0 + k * MROW, MROW)

    def _zero_fill(buf):
        def _zrow(i, _):
            for j in range(DN // 16):
                buf[i, pl.ds(j * 16, 16)] = zvec
            return 0
        lax.fori_loop(0, MROW, _zrow, 0)

    _zero_fill(msg[0])
    _zero_fill(msg[1])
    my_node0 = s * NODES_PER_TILE
    for k in range(NODES_PER_TILE // MROW):
        pltpu.sync_copy(msg[0].at[pl.ds(0, MROW)],
                        aggr_ref.at[pl.ds(my_node0 + k * MROW, MROW)])
    for b in range(2):
        for off in (0, 16, 24):
            sidx[b][pl.ds(off, 16)] = zivec
    plsc.subcore_barrier()

    for b in range(2):
        pltpu.async_copy(msg[b], aggr_ref.at[sidx[b]], sem_s, add=True)

    pltpu.sync_copy(src_hbm.at[_esl(0)], idx_s[0])
    pltpu.sync_copy(dst_hbm.at[_esl(0)], idx_d[0])
    pltpu.async_copy(ea_hbm.at[_esl(0)], ea[0], sem_d)
    pltpu.async_copy(x_hbm.at[idx_s[0]], xg[0], sem_d)
    pltpu.async_copy(src_hbm.at[_esl(1)], idx_s[1], sem_i)
    pltpu.async_copy(dst_hbm.at[_esl(1)], idx_d[1], sem_i)

    def _body(j, b):
        nb = 1 - b
        jn = jnp.minimum(j + 1, M_CH - 1)
        jnn = jnp.minimum(j + 2, M_CH - 1)
        pltpu.make_async_copy(src_hbm.at[_esl(jn)], idx_s[nb], sem_i).wait()
        pltpu.make_async_copy(dst_hbm.at[_esl(jn)], idx_d[nb], sem_i).wait()
        pltpu.async_copy(ea_hbm.at[_esl(jn)], ea[nb], sem_d)
        pltpu.async_copy(x_hbm.at[idx_s[nb]], xg[nb], sem_d)
        pltpu.make_async_copy(msg[b], aggr_ref.at[sidx[b]], sem_s).wait()
        pltpu.make_async_copy(ea_hbm.at[_esl(j)], ea[b], sem_d).wait()
        pltpu.make_async_copy(x_hbm.at[idx_s[b]], xg[b], sem_d).wait()

        def _edge(e, _):
            for jj in range(DN // 16):
                sl = pl.ds(jj * 16, 16)
                msg[b][e, sl] = jnp.maximum(ea[b][e, sl] + xg[b][e, sl], 0.0)
            return 0
        lax.fori_loop(0, MROW, _edge, 0)

        for off in (0, 16, 24):
            sl = pl.ds(off, 16)
            sidx[b][sl] = idx_d[b][sl]
        pltpu.async_copy(src_hbm.at[_esl(jnn)], idx_s[b], sem_i)
        pltpu.async_copy(dst_hbm.at[_esl(jnn)], idx_d[b], sem_i)
        pltpu.async_copy(msg[b], aggr_ref.at[sidx[b]], sem_s, add=True)

    def _pair(g, _):
        _body(2 * g, 0)
        _body(2 * g + 1, 1)
        return 0

    lax.fori_loop(0, M_PAIRS, _pair, 0)

    pltpu.make_async_copy(src_hbm.at[_esl(0)], idx_s[1], sem_i).wait()
    pltpu.make_async_copy(dst_hbm.at[_esl(0)], idx_d[1], sem_i).wait()
    pltpu.make_async_copy(ea_hbm.at[_esl(0)], ea[0], sem_d).wait()
    pltpu.make_async_copy(x_hbm.at[idx_s[0]], xg[0], sem_d).wait()
    pltpu.make_async_copy(msg[0], aggr_ref.at[sidx[0]], sem_s).wait()
    pltpu.make_async_copy(msg[1], aggr_ref.at[sidx[1]], sem_s).wait()
    plsc.subcore_barrier()

    for k in range(NODES_PER_TILE // MROW):
        r = my_node0 + k * MROW
        pltpu.sync_copy(aggr_ref.at[pl.ds(r, MROW)], msg[0].at[pl.ds(0, MROW)])
        pltpu.sync_copy(msg[0].at[pl.ds(0, MROW)], out_hbm.at[c, pl.ds(r, MROW)])


@functools.partial(
    pl.kernel,
    out_type=jax.ShapeDtypeStruct((N_EDGES * DE,), jnp.float32),
    mesh=_mesh,
    scratch_types=[
        [pltpu.VMEM((CH,), jnp.int32)] * 2,
        [pltpu.VMEM((CH,), jnp.int32)] * 2,
        [pltpu.VMEM((CH, DE), jnp.float32)] * 2,
        [pltpu.VMEM((CH, DE), jnp.float32)] * 2,
        [pltpu.VMEM((CH * DE,), jnp.float32)] * 2,
        [pltpu.VMEM((CH * DE,), jnp.float32)] * 2,
        pltpu.SemaphoreType.DMA,
        pltpu.SemaphoreType.DMA,
        pltpu.SemaphoreType.DMA,
    ],
    compiler_params=pltpu.CompilerParams(use_tc_tiling_on_sc=False),
)
def _edgegather(src_hbm, dst_hbm, p_hbm, q_hbm, r_hbm, t_hbm,
                idx_s, idx_d, pg, qg, rr, rb, sem_i, sem_d, sem_o):
    c = lax.axis_index("c")
    s = lax.axis_index("s")
    wid = c * NS + s
    e0 = wid * E_PER_TILE

    def _esl(k):
        return pl.ds(e0 + k * CH, CH)

    def _tsl(k):
        return pl.ds((e0 + k * CH) * DE, CH * DE)

    def _start_data(k, b):
        pltpu.async_copy(r_hbm.at[_tsl(k)], rr[b], sem_d)
        for j in range(CH_ROWS):
            sl = pl.ds(j * ROW, ROW)
            pltpu.async_copy(p_hbm.at[idx_s[b].at[sl]], pg[b].at[sl], sem_d)
            pltpu.async_copy(q_hbm.at[idx_d[b].at[sl]], qg[b].at[sl], sem_d)

    def _wait_data(k, b):
        pltpu.make_async_copy(r_hbm.at[_tsl(k)], rr[b], sem_d).wait()
        for j in range(CH_ROWS):
            sl = pl.ds(j * ROW, ROW)
            pltpu.make_async_copy(p_hbm.at[idx_s[b].at[sl]],
                                  pg[b].at[sl], sem_d).wait()
            pltpu.make_async_copy(q_hbm.at[idx_d[b].at[sl]],
                                  qg[b].at[sl], sem_d).wait()

    def _body(j, b, first=False):
        nb = 1 - b
        jn = jnp.minimum(j + 1, N_CH - 1)
        jnn = jnp.minimum(j + 2, N_CH - 1)
        pltpu.make_async_copy(src_hbm.at[_esl(jn)], idx_s[nb], sem_i).wait()
        pltpu.make_async_copy(dst_hbm.at[_esl(jn)], idx_d[nb], sem_i).wait()
        if not first:
            pltpu.make_async_copy(rb[nb], t_hbm.at[_tsl(j)], sem_o).wait()
        _start_data(jn, nb)
        _wait_data(j, b)

        def _edge(e, _):
            sl = pl.ds(e * DE, DE)
            rb[b][sl] = jnp.maximum(rr[b][sl] + pg[b][e] + qg[b][e], 0.0)
            return 0

        lax.fori_loop(0, CH, _edge, 0)
        pltpu.async_copy(src_hbm.at[_esl(jnn)], idx_s[b], sem_i)
        pltpu.async_copy(dst_hbm.at[_esl(jnn)], idx_d[b], sem_i)
        pltpu.async_copy(rb[b], t_hbm.at[_tsl(j)], sem_o)

    pltpu.sync_copy(src_hbm.at[_esl(0)], idx_s[0])
    pltpu.sync_copy(dst_hbm.at[_esl(0)], idx_d[0])
    _start_data(0, 0)
    pltpu.async_copy(src_hbm.at[_esl(1)], idx_s[1], sem_i)
    pltpu.async_copy(dst_hbm.at[_esl(1)], idx_d[1], sem_i)

    _body(0, 0, first=True)

    def _pair(g, _):
        _body(2 * g + 1, 1)
        _body(2 * g + 2, 0)
        return 0

    lax.fori_loop(0, (N_CH - 1) // 2, _pair, 0)

    pltpu.make_async_copy(src_hbm.at[_esl(0)], idx_s[0], sem_i).wait()
    pltpu.make_async_copy(dst_hbm.at[_esl(0)], idx_d[0], sem_i).wait()
    _wait_data(N_CH - 1, 1)
    pltpu.make_async_copy(rb[0], t_hbm.at[_tsl(N_CH - 1)], sem_o).wait()


_EBLK = 8000
_EGRID = N_EDGES // _EBLK
_OBLK = 16000
_OGRID = N_EDGES // _OBLK


_E8BLK = 1000


def _ea_body(e_ref, w0_ref, b0_ref, ea0_ref):
    ea0_ref[...] = jnp.dot(e_ref[...], w0_ref[...],
                           preferred_element_type=jnp.float32) + b0_ref[...]


def _ea_proj(edge_attr, w0, b0):
    full = lambda shp: pl.BlockSpec(shp, lambda i: (0, 0))
    return pl.pallas_call(
        _ea_body,
        grid=(_EGRID,),
        in_specs=[
            pl.BlockSpec((_EBLK, DE), lambda i: (i, 0)),
            full((DE, DN)), full((1, DN)),
        ],
        out_specs=pl.BlockSpec((_EBLK, DN), lambda i: (i, 0)),
        out_shape=jax.ShapeDtypeStruct((N_EDGES, DN), jnp.float32),
    )(edge_attr, w0, b0)


def _ea_r_body(e_ref, e8_ref, w1_ref, b1_ref, w8c_ref, b8c_ref,
               ea1_ref, r8_ref):
    ea1_ref[...] = jnp.dot(e_ref[...], w1_ref[...],
                           preferred_element_type=jnp.float32) + b1_ref[...]
    r8_ref[...] = jnp.dot(e8_ref[...], w8c_ref[...],
                          preferred_element_type=jnp.float32) + b8c_ref[...]


def _ea_r_proj(edge_attr, ea8, w1, b1, w8c, b8c):
    full = lambda shp: pl.BlockSpec(shp, lambda i: (0, 0))
    return pl.pallas_call(
        _ea_r_body,
        grid=(_EGRID,),
        in_specs=[
            pl.BlockSpec((_EBLK, DE), lambda i: (i, 0)),
            pl.BlockSpec((_E8BLK, DN), lambda i: (i, 0)),
            full((DE, DN)), full((1, DN)),
            full((DN, DN)), full((1, DN)),
        ],
        out_specs=[
            pl.BlockSpec((_EBLK, DN), lambda i: (i, 0)),
            pl.BlockSpec((_E8BLK, DN), lambda i: (i, 0)),
        ],
        out_shape=[
            jax.ShapeDtypeStruct((N_EDGES, DN), jnp.float32),
            jax.ShapeDtypeStruct((N_EDGES * DE // DN, DN), jnp.float32),
        ],
    )(edge_attr, ea8, w1, b1, w8c, b8c)


def _node_body(x_ref, a_ref, w1_ref, b1_ref, w2_ref, b2_ref, gb_ref,
               epsb_ref, wp_ref, wq_ref, xn_ref, pp_ref, qq_ref):
    x = x_ref[...]
    aggr = (a_ref[0] + a_ref[1])[:N_NODES]
    h = epsb_ref[...] * x + aggr
    h = jnp.maximum(jnp.dot(h, w1_ref[...],
                            preferred_element_type=jnp.float32) + b1_ref[...],
                    0.0)
    h = jnp.dot(h, w2_ref[...],
                preferred_element_type=jnp.float32) + b2_ref[...]
    mean = jnp.mean(h, axis=0, keepdims=True)
    cent = h - mean
    var = jnp.mean(cent * cent, axis=0, keepdims=True)
    bn = gb_ref[0:1, :] * cent * lax.rsqrt(var + 1e-5) + gb_ref[1:2, :]
    xn = (x + jnp.maximum(bn, 0.0)) * 0.5
    xn_ref[...] = xn
    pp_ref[...] = jnp.dot(xn, wp_ref[...], preferred_element_type=jnp.float32)
    qq_ref[...] = jnp.dot(xn, wq_ref[...], preferred_element_type=jnp.float32)


def _node(x, a2, w1, b1, w2, b2, gb, epsb, wp, wq):
    return pl.pallas_call(
        _node_body,
        out_shape=[
            jax.ShapeDtypeStruct((N_NODES, DN), jnp.float32),
            jax.ShapeDtypeStruct((N_NODES, DE), jnp.float32),
            jax.ShapeDtypeStruct((N_NODES, DE), jnp.float32),
        ],
    )(x, a2, w1, b1, w2, b2, gb, epsb, wp, wq)


_T8ROWS = N_EDGES * DE // DN
_T8BLK = _T8ROWS // _OGRID


def _eout_body(t_ref, ea8_ref, w_ref, b_ref, o_ref):
    mlp = jnp.dot(t_ref[...], w_ref[...],
                  preferred_element_type=jnp.float32) + b_ref[...]
    o_ref[...] = ea8_ref[...] + mlp * 0.5


def _eout(t8, ea8, w8, b8):
    full = lambda shp: pl.BlockSpec(shp, lambda i: (0, 0))
    return pl.pallas_call(
        _eout_body,
        grid=(_OGRID,),
        in_specs=[
            pl.BlockSpec((_T8BLK, DN), lambda i: (i, 0)),
            pl.BlockSpec((_T8BLK, DN), lambda i: (i, 0)),
            full((DN, DN)), full((1, DN)),
        ],
        out_specs=pl.BlockSpec((_T8BLK, DN), lambda i: (i, 0)),
        out_shape=jax.ShapeDtypeStruct((_T8ROWS, DN), jnp.float32),
    )(t8, ea8, w8, b8)


def kernel(x, edge_index, edge_attr, params):
    src_i = edge_index[0].astype(jnp.int32)
    dst_i = edge_index[1].astype(jnp.int32)
    p0, p1 = params[0], params[1]

    wp = p1['Wm1'][0:DN]
    wq = p1['Wm1'][DN:2 * DN]
    wc = p1['Wm1'][2 * DN:]

    ea8 = edge_attr.reshape(_T8ROWS, DN)
    w8c = jnp.kron(jnp.eye(8, dtype=jnp.float32), wc)
    b8c = jnp.tile(p1['bm1'], 8).reshape(1, DN)
    ea0 = _ea_proj(edge_attr, p0['We'], p0['be'].reshape(1, DN))
    ea1, rmat8 = _ea_r_proj(edge_attr, ea8, p1['We'], p1['be'].reshape(1, DN),
                            w8c, b8c)

    ones = jnp.ones((1, DN), jnp.float32)
    for p, ea in ((p0, ea0), (p1, ea1)):
        a2 = _msgpass(src_i, dst_i, ea, x)
        gb = jnp.stack([p['bn_gamma'], p['bn_beta']])
        epsb = (1.0 + p['eps']) * ones
        x, pp, qq = _node(x, a2, p['W1'], p['b1'].reshape(1, DN),
                          p['W2'], p['b2'].reshape(1, DN), gb, epsb, wp, wq)

    r_flat = rmat8.reshape(N_EDGES * DE)
    t8 = _edgegather(src_i, dst_i, pp, qq, r_flat).reshape(_T8ROWS, DN)
    w8 = jnp.kron(jnp.eye(8, dtype=jnp.float32), p1['Wm2'])
    b8 = jnp.tile(p1['bm2'], 8).reshape(1, DN)
    e_out = _eout(t8, ea8, w8, b8).reshape(N_EDGES, DE)
    return (x, e_out)


# speedup vs baseline: 4.3889x; 1.0087x over previous
import functools

import jax
import jax.numpy as jnp
from jax import lax
from jax.experimental import pallas as pl
from jax.experimental.pallas import tpu as pltpu
from jax.experimental.pallas import tpu_sc as plsc

N_NODES = 10000
N_EDGES = 320000
DN = 128
DE = 16

NC, NS = 2, 16
NW = NC * NS
E_PER_TILE = N_EDGES // NW
ROW = 80
CH_ROWS = 5
CH = ROW * CH_ROWS
N_CH = E_PER_TILE // CH
MROW = 40
M_CH = E_PER_TILE // MROW
M_PAIRS = M_CH // 2
N_PAD = 10240
NODES_PER_TILE = N_PAD // NS
ZR = 64

_mesh = plsc.VectorSubcoreMesh(core_axis_name="c", subcore_axis_name="s")


@functools.partial(
    pl.kernel,
    out_type=jax.ShapeDtypeStruct((NC, N_PAD, DN), jnp.float32),
    mesh=_mesh,
    scratch_types=[
        [pltpu.VMEM((MROW,), jnp.int32)] * 2,
        [pltpu.VMEM((MROW,), jnp.int32)] * 2,
        [pltpu.VMEM((MROW,), jnp.int32)] * 2,
        [pltpu.VMEM((MROW, DN), jnp.float32)] * 2,
        [pltpu.VMEM((MROW, DN), jnp.float32)] * 2,
        [pltpu.VMEM((MROW, DN), jnp.float32)] * 2,
        pltpu.VMEM_SHARED((N_PAD, DN), jnp.float32),
        pltpu.SemaphoreType.DMA,
        pltpu.SemaphoreType.DMA,
        pltpu.SemaphoreType.DMA,
    ],
)
def _msgpass(src_hbm, dst_hbm, ea_hbm, x_hbm, out_hbm,
             idx_s, idx_d, sidx, xg, ea, msg, aggr_ref, sem_i, sem_d, sem_s):
    c = lax.axis_index("c")
    s = lax.axis_index("s")
    wid = c * NS + s
    e0 = wid * E_PER_TILE
    zvec = jnp.zeros((16,), jnp.float32)
    zivec = jnp.full((16,), N_PAD - 8, jnp.int32)

    def _esl(k):
        return pl.ds(e0 + k * MROW, MROW)

    def _zero_fill(buf):
        def _zrow(i, _):
            for j in range(DN // 16):
                buf[i, pl.ds(j * 16, 16)] = zvec
            return 0
        lax.fori_loop(0, MROW, _zrow, 0)

    _zero_fill(msg[0])
    _zero_fill(msg[1])
    my_node0 = s * NODES_PER_TILE
    for k in range(NODES_PER_TILE // MROW):
        pltpu.sync_copy(msg[0].at[pl.ds(0, MROW)],
                        aggr_ref.at[pl.ds(my_node0 + k * MROW, MROW)])
    for b in range(2):
        for off in (0, 16, 24):
            sidx[b][pl.ds(off, 16)] = zivec
    plsc.subcore_barrier()

    for b in range(2):
        pltpu.async_copy(msg[b], aggr_ref.at[sidx[b]], sem_s, add=True)

    pltpu.sync_copy(src_hbm.at[_esl(0)], idx_s[0])
    pltpu.sync_copy(dst_hbm.at[_esl(0)], idx_d[0])
    pltpu.async_copy(ea_hbm.at[_esl(0)], ea[0], sem_d)
    pltpu.async_copy(x_hbm.at[idx_s[0]], xg[0], sem_d)
    pltpu.async_copy(src_hbm.at[_esl(1)], idx_s[1], sem_i)
    pltpu.async_copy(dst_hbm.at[_esl(1)], idx_d[1], sem_i)

    def _body(j, b):
        nb = 1 - b
        jn = jnp.minimum(j + 1, M_CH - 1)
        jnn = jnp.minimum(j + 2, M_CH - 1)
        pltpu.make_async_copy(src_hbm.at[_esl(jn)], idx_s[nb], sem_i).wait()
        pltpu.make_async_copy(dst_hbm.at[_esl(jn)], idx_d[nb], sem_i).wait()
        pltpu.async_copy(ea_hbm.at[_esl(jn)], ea[nb], sem_d)
        pltpu.async_copy(x_hbm.at[idx_s[nb]], xg[nb], sem_d)
        pltpu.make_async_copy(msg[b], aggr_ref.at[sidx[b]], sem_s).wait()
        pltpu.make_async_copy(ea_hbm.at[_esl(j)], ea[b], sem_d).wait()
        pltpu.make_async_copy(x_hbm.at[idx_s[b]], xg[b], sem_d).wait()

        def _edge(e, _):
            for jj in range(DN // 16):
                sl = pl.ds(jj * 16, 16)
                msg[b][e, sl] = jnp.maximum(ea[b][e, sl] + xg[b][e, sl], 0.0)
            return 0
        lax.fori_loop(0, MROW, _edge, 0)

        for off in (0, 16, 24):
            sl = pl.ds(off, 16)
            sidx[b][sl] = idx_d[b][sl]
        pltpu.async_copy(src_hbm.at[_esl(jnn)], idx_s[b], sem_i)
        pltpu.async_copy(dst_hbm.at[_esl(jnn)], idx_d[b], sem_i)
        pltpu.async_copy(msg[b], aggr_ref.at[sidx[b]], sem_s, add=True)

    def _pair(g, _):
        _body(2 * g, 0)
        _body(2 * g + 1, 1)
        return 0

    lax.fori_loop(0, M_PAIRS, _pair, 0)

    pltpu.make_async_copy(src_hbm.at[_esl(0)], idx_s[1], sem_i).wait()
    pltpu.make_async_copy(dst_hbm.at[_esl(0)], idx_d[1], sem_i).wait()
    pltpu.make_async_copy(ea_hbm.at[_esl(0)], ea[0], sem_d).wait()
    pltpu.make_async_copy(x_hbm.at[idx_s[0]], xg[0], sem_d).wait()
    pltpu.make_async_copy(msg[0], aggr_ref.at[sidx[0]], sem_s).wait()
    pltpu.make_async_copy(msg[1], aggr_ref.at[sidx[1]], sem_s).wait()
    plsc.subcore_barrier()

    for k in range(NODES_PER_TILE // MROW):
        r = my_node0 + k * MROW
        pltpu.sync_copy(aggr_ref.at[pl.ds(r, MROW)], msg[0].at[pl.ds(0, MROW)])
        pltpu.sync_copy(msg[0].at[pl.ds(0, MROW)], out_hbm.at[c, pl.ds(r, MROW)])


@functools.partial(
    pl.kernel,
    out_type=jax.ShapeDtypeStruct((N_EDGES * DE,), jnp.float32),
    mesh=_mesh,
    scratch_types=[
        [pltpu.VMEM((CH,), jnp.int32)] * 2,
        [pltpu.VMEM((CH,), jnp.int32)] * 2,
        [pltpu.VMEM((CH, DE), jnp.float32)] * 2,
        [pltpu.VMEM((CH, DE), jnp.float32)] * 2,
        [pltpu.VMEM((CH * DE,), jnp.float32)] * 2,
        [pltpu.VMEM((CH * DE,), jnp.float32)] * 2,
        pltpu.SemaphoreType.DMA,
        pltpu.SemaphoreType.DMA,
        pltpu.SemaphoreType.DMA,
    ],
    compiler_params=pltpu.CompilerParams(use_tc_tiling_on_sc=False),
)
def _edgegather(src_hbm, dst_hbm, p_hbm, q_hbm, r_hbm, t_hbm,
                idx_s, idx_d, pg, qg, rr, rb, sem_i, sem_d, sem_o):
    c = lax.axis_index("c")
    s = lax.axis_index("s")
    wid = c * NS + s
    e0 = wid * E_PER_TILE

    def _esl(k):
        return pl.ds(e0 + k * CH, CH)

    def _tsl(k):
        return pl.ds((e0 + k * CH) * DE, CH * DE)

    def _start_data(k, b):
        pltpu.async_copy(r_hbm.at[_tsl(k)], rr[b], sem_d)
        for j in range(CH_ROWS):
            sl = pl.ds(j * ROW, ROW)
            pltpu.async_copy(p_hbm.at[idx_s[b].at[sl]], pg[b].at[sl], sem_d)
            pltpu.async_copy(q_hbm.at[idx_d[b].at[sl]], qg[b].at[sl], sem_d)

    def _wait_data(k, b):
        pltpu.make_async_copy(r_hbm.at[_tsl(k)], rr[b], sem_d).wait()
        for j in range(CH_ROWS):
            sl = pl.ds(j * ROW, ROW)
            pltpu.make_async_copy(p_hbm.at[idx_s[b].at[sl]],
                                  pg[b].at[sl], sem_d).wait()
            pltpu.make_async_copy(q_hbm.at[idx_d[b].at[sl]],
                                  qg[b].at[sl], sem_d).wait()

    def _body(j, b, first=False):
        nb = 1 - b
        jn = jnp.minimum(j + 1, N_CH - 1)
        jnn = jnp.minimum(j + 2, N_CH - 1)
        pltpu.make_async_copy(src_hbm.at[_esl(jn)], idx_s[nb], sem_i).wait()
        pltpu.make_async_copy(dst_hbm.at[_esl(jn)], idx_d[nb], sem_i).wait()
        if not first:
            pltpu.make_async_copy(rb[nb], t_hbm.at[_tsl(j)], sem_o).wait()
        _start_data(jn, nb)
        _wait_data(j, b)

        def _edge(e, _):
            sl = pl.ds(e * DE, DE)
            rb[b][sl] = jnp.maximum(rr[b][sl] + pg[b][e] + qg[b][e], 0.0)
            return 0

        lax.fori_loop(0, CH, _edge, 0)
        pltpu.async_copy(src_hbm.at[_esl(jnn)], idx_s[b], sem_i)
        pltpu.async_copy(dst_hbm.at[_esl(jnn)], idx_d[b], sem_i)
        pltpu.async_copy(rb[b], t_hbm.at[_tsl(j)], sem_o)

    pltpu.sync_copy(src_hbm.at[_esl(0)], idx_s[0])
    pltpu.sync_copy(dst_hbm.at[_esl(0)], idx_d[0])
    _start_data(0, 0)
    pltpu.async_copy(src_hbm.at[_esl(1)], idx_s[1], sem_i)
    pltpu.async_copy(dst_hbm.at[_esl(1)], idx_d[1], sem_i)

    _body(0, 0, first=True)

    def _pair(g, _):
        _body(2 * g + 1, 1)
        _body(2 * g + 2, 0)
        return 0

    lax.fori_loop(0, (N_CH - 1) // 2, _pair, 0)

    pltpu.make_async_copy(src_hbm.at[_esl(0)], idx_s[0], sem_i).wait()
    pltpu.make_async_copy(dst_hbm.at[_esl(0)], idx_d[0], sem_i).wait()
    _wait_data(N_CH - 1, 1)
    pltpu.make_async_copy(rb[0], t_hbm.at[_tsl(N_CH - 1)], sem_o).wait()


_EBLK = 8000
_EGRID = N_EDGES // _EBLK
_OBLK = 16000
_OGRID = N_EDGES // _OBLK


_E8BLK = 1000


def _ea0_body(e8_ref, w0_ref, b0_ref, ea0_ref):
    e8 = e8_ref[...]
    for u in range(8):
        ea0_ref[u] = jnp.dot(e8, w0_ref[u],
                             preferred_element_type=jnp.float32) + b0_ref[...]


def _ea0_proj(ea8bf, w0s, b0):
    full = lambda shp: pl.BlockSpec(shp, lambda i: tuple([0] * len(shp)))
    return pl.pallas_call(
        _ea0_body,
        grid=(_EGRID,),
        in_specs=[
            pl.BlockSpec((_E8BLK, DN), lambda i: (i, 0)),
            full((8, DN, DN)), full((1, DN)),
        ],
        out_specs=pl.BlockSpec((8, _E8BLK, DN), lambda i: (0, i, 0)),
        out_shape=jax.ShapeDtypeStruct((8, N_EDGES // 8, DN), jnp.float32),
    )(ea8bf, w0s, b0)


def _ea_r_body(e8_ref, e8f_ref, w1_ref, b1_ref, w8c_ref, b8c_ref,
               ea1_ref, r8_ref):
    e8 = e8_ref[...]
    for u in range(8):
        ea1_ref[u] = jnp.dot(e8, w1_ref[u],
                             preferred_element_type=jnp.float32) + b1_ref[...]
    r8_ref[...] = jnp.dot(e8f_ref[...], w8c_ref[...],
                          preferred_element_type=jnp.float32) + b8c_ref[...]


def _ea_r_proj(ea8bf, ea8, w1s, b1, w8c, b8c):
    full = lambda shp: pl.BlockSpec(shp, lambda i: tuple([0] * len(shp)))
    return pl.pallas_call(
        _ea_r_body,
        grid=(_EGRID,),
        in_specs=[
            pl.BlockSpec((_E8BLK, DN), lambda i: (i, 0)),
            pl.BlockSpec((_E8BLK, DN), lambda i: (i, 0)),
            full((8, DN, DN)), full((1, DN)),
            full((DN, DN)), full((1, DN)),
        ],
        out_specs=[
            pl.BlockSpec((8, _E8BLK, DN), lambda i: (0, i, 0)),
            pl.BlockSpec((_E8BLK, DN), lambda i: (i, 0)),
        ],
        out_shape=[
            jax.ShapeDtypeStruct((8, N_EDGES // 8, DN), jnp.float32),
            jax.ShapeDtypeStruct((N_EDGES * DE // DN, DN), jnp.float32),
        ],
    )(ea8bf, ea8, w1s, b1, w8c, b8c)


def _node_body(x_ref, a_ref, w1_ref, b1_ref, w2_ref, b2_ref, gb_ref,
               epsb_ref, wp_ref, wq_ref, xn_ref, pp_ref, qq_ref):
    x = x_ref[...]
    aggr = (a_ref[0] + a_ref[1])[:N_NODES]
    h = epsb_ref[...] * x + aggr
    h = jnp.maximum(jnp.dot(h, w1_ref[...],
                            preferred_element_type=jnp.float32) + b1_ref[...],
                    0.0)
    h = jnp.dot(h, w2_ref[...],
                preferred_element_type=jnp.float32) + b2_ref[...]
    mean = jnp.mean(h, axis=0, keepdims=True)
    cent = h - mean
    var = jnp.mean(cent * cent, axis=0, keepdims=True)
    bn = gb_ref[0:1, :] * cent * lax.rsqrt(var + 1e-5) + gb_ref[1:2, :]
    xn = (x + jnp.maximum(bn, 0.0)) * 0.5
    xn_ref[...] = xn
    pp_ref[...] = jnp.dot(xn, wp_ref[...], preferred_element_type=jnp.float32)
    qq_ref[...] = jnp.dot(xn, wq_ref[...], preferred_element_type=jnp.float32)


def _node(x, a2, w1, b1, w2, b2, gb, epsb, wp, wq):
    return pl.pallas_call(
        _node_body,
        out_shape=[
            jax.ShapeDtypeStruct((N_NODES, DN), jnp.float32),
            jax.ShapeDtypeStruct((N_NODES, DE), jnp.float32),
            jax.ShapeDtypeStruct((N_NODES, DE), jnp.float32),
        ],
    )(x, a2, w1, b1, w2, b2, gb, epsb, wp, wq)


_T8ROWS = N_EDGES * DE // DN
_T8BLK = _T8ROWS // _OGRID


def _eout_body(t_ref, ea8_ref, w_ref, b_ref, o_ref):
    mlp = jnp.dot(t_ref[...], w_ref[...],
                  preferred_element_type=jnp.float32) + b_ref[...]
    o_ref[...] = ea8_ref[...] + mlp * 0.5


def _eout(t8, ea8, w8, b8):
    full = lambda shp: pl.BlockSpec(shp, lambda i: (0, 0))
    return pl.pallas_call(
        _eout_body,
        grid=(_OGRID,),
        in_specs=[
            pl.BlockSpec((_T8BLK, DN), lambda i: (i, 0)),
            pl.BlockSpec((_T8BLK, DN), lambda i: (i, 0)),
            full((DN, DN)), full((1, DN)),
        ],
        out_specs=pl.BlockSpec((_T8BLK, DN), lambda i: (i, 0)),
        out_shape=jax.ShapeDtypeStruct((_T8ROWS, DN), jnp.float32),
    )(t8, ea8, w8, b8)


def kernel(x, edge_index, edge_attr, params):
    src_i = edge_index[0].astype(jnp.int32)
    dst_i = edge_index[1].astype(jnp.int32)
    p0, p1 = params[0], params[1]

    wp = p1['Wm1'][0:DN]
    wq = p1['Wm1'][DN:2 * DN]
    wc = p1['Wm1'][2 * DN:]

    ea8 = edge_attr.reshape(_T8ROWS, DN)
    ea8bf = ea8.astype(jnp.bfloat16)
    w8c = jnp.kron(jnp.eye(8, dtype=jnp.float32), wc)
    b8c = jnp.tile(p1['bm1'], 8).reshape(1, DN)

    def _embed8(w):
        z = jnp.zeros((DN, DN), jnp.float32)
        return jnp.stack([z.at[u * DE:(u + 1) * DE].set(w)
                          for u in range(8)]).astype(jnp.bfloat16)

    src_p = src_i.reshape(N_EDGES // 8, 8).T.reshape(-1)
    dst_p = dst_i.reshape(N_EDGES // 8, 8).T.reshape(-1)

    ea0 = _ea0_proj(ea8bf, _embed8(p0['We']),
                    p0['be'].reshape(1, DN)).reshape(N_EDGES, DN)
    ea1, rmat8 = _ea_r_proj(ea8bf, ea8, _embed8(p1['We']),
                            p1['be'].reshape(1, DN), w8c, b8c)
    ea1 = ea1.reshape(N_EDGES, DN)

    ones = jnp.ones((1, DN), jnp.float32)
    for p, ea in ((p0, ea0), (p1, ea1)):
        a2 = _msgpass(src_p, dst_p, ea, x)
        gb = jnp.stack([p['bn_gamma'], p['bn_beta']])
        epsb = (1.0 + p['eps']) * ones
        x, pp, qq = _node(x, a2, p['W1'], p['b1'].reshape(1, DN),
                          p['W2'], p['b2'].reshape(1, DN), gb, epsb, wp, wq)

    r_flat = rmat8.reshape(N_EDGES * DE)
    t8 = _edgegather(src_i, dst_i, pp, qq, r_flat).reshape(_T8ROWS, DN)
    w8 = jnp.kron(jnp.eye(8, dtype=jnp.float32), p1['Wm2'])
    b8 = jnp.tile(p1['bm2'], 8).reshape(1, DN)
    e_out = _eout(t8, ea8, w8, b8).reshape(N_EDGES, DE)
    return (x, e_out)
